# baseline (speedup 1.0000x reference)
import functools

import jax
import jax.numpy as jnp
from jax import lax
from jax.experimental import pallas as pl
from jax.experimental.pallas import tpu as pltpu
from jax.experimental.pallas import tpu_sc as plsc

_N = 50000
_E = 800000
_G = 256
_NF = 21
_DIM = 256

_NC = 2
_NS = 16
_NW = _NC * _NS

_CH = 32
_NCH = _DIM // _CH
_NR = 50176
_W = 1024
_KI = _W // 128
_NWIN = 25
_EW = _NWIN * _W
_EPAD = _NW * _EW
_GS = 272
_PGRP = _NR // 1024
_BN = 512
_NBLK = _NR // _BN
_BE = 2048


def _mesh():
    return plsc.VectorSubcoreMesh(core_axis_name="c", subcore_axis_name="s",
                                  num_cores=_NC, num_subcores=_NS)


def _make_scatter(nchunk):
    acc_rows_per_tile = _NR // _NS
    n_zero = acc_rows_per_tile // 64
    _Q = _W // 4

    def body(xt, e, srcn, dstr, parts, acc, sidx, didx, xga, xgb, eb, sema,
             semb):
        cid = lax.axis_index("c")
        sid = lax.axis_index("s")
        wid = sid * _NC + cid
        bufs = (xga, xgb)
        sems = (sema, semb)

        def gather(h, buf, sem):
            return [pltpu.async_copy(xt.at[sidx.at[h * 2 + j]],
                                     buf.at[pl.ds(j * 128, 128)], sem)
                    for j in range(2)]

        @pl.loop(0, nchunk)
        def _chunk(ch):
            @pl.loop(0, 64)
            def _fill_z(r):
                z = jnp.zeros((16,), jnp.float32)
                xga[r, 0:16] = z
                xga[r, 16:32] = z

            @pl.loop(0, n_zero)
            def _zero(i):
                pltpu.sync_copy(
                    xga.at[pl.ds(0, 64)],
                    acc.at[pl.ds(pl.multiple_of(
                        sid * acc_rows_per_tile + i * 64, 64), 64)])

            plsc.subcore_barrier()

            @pl.loop(0, _NWIN)
            def _win(w):
                base = pl.multiple_of(wid * _EW + w * _W, _W)
                r0 = pl.multiple_of(ch * (_EPAD // 128) + base // 128, 8)
                pltpu.sync_copy(srcn.at[pl.ds(r0, _KI)], sidx)
                pltpu.sync_copy(dstr.at[pl.ds(pl.multiple_of(base // 128, 8),
                                              _KI)], didx)
                descs = {0: gather(0, bufs[0], sems[0])}
                for h in range(4):
                    p = h % 2
                    xg = bufs[p]
                    for d in descs.pop(p):
                        d.wait()
                    if h < 3:
                        descs[1 - p] = gather(h + 1, bufs[1 - p],
                                              sems[1 - p])
                    pltpu.sync_copy(
                        e.at[pl.ds(pl.multiple_of(
                            ch * _EPAD + base + h * _Q, _Q), _Q)], eb)

                    @pl.loop(0, _Q, unroll=8)
                    def _msg(i):
                        a = xg[i, 0:16] + eb[i, 0:16]
                        xg[i, 0:16] = jnp.maximum(a, 0.0)
                        b = xg[i, 16:32] + eb[i, 16:32]
                        xg[i, 16:32] = jnp.maximum(b, 0.0)

                    for j in range(2):
                        pltpu.sync_copy(xg.at[pl.ds(j * 128, 128)],
                                        acc.at[didx.at[h * 2 + j]],
                                        add=True)

            plsc.subcore_barrier()

            row = pl.multiple_of(sid * acc_rows_per_tile, 64)
            pltpu.sync_copy(
                acc.at[pl.ds(row, acc_rows_per_tile)],
                parts.at[cid, pl.ds(pl.multiple_of(ch * _NR + row, 64),
                                    acc_rows_per_tile)])

            plsc.subcore_barrier()

    return pl.kernel(
        body,
        out_type=jax.ShapeDtypeStruct((_NC, nchunk * _NR, _CH), jnp.float32),
        mesh=_mesh(),
        compiler_params=pltpu.CompilerParams(use_tc_tiling_on_sc=False),
        scratch_types=[
            pltpu.VMEM_SHARED((_NR, _CH), jnp.float32),
            pltpu.VMEM((_KI, 128), jnp.int32),
            pltpu.VMEM((_KI, 128), jnp.int32),
            pltpu.VMEM((_Q, _CH), jnp.float32),
            pltpu.VMEM((_Q, _CH), jnp.float32),
            pltpu.VMEM((_Q, _CH), jnp.float32),
            pltpu.SemaphoreType.DMA,
            pltpu.SemaphoreType.DMA,
        ],
    )


def _make_pool():
    arows = _NCH * _GS
    rows_per_tile = arows // _NS

    def body(xt, b8, pparts, acc, bidx, xb, zb, sem):
        del sem
        cid = lax.axis_index("c")
        sid = lax.axis_index("s")
        wid = sid * _NC + cid

        @pl.loop(0, rows_per_tile)
        def _fill_zb(r):
            z = jnp.zeros((16,), jnp.float32)
            zb[r, 0:16] = z
            zb[r, 16:32] = z

        pltpu.sync_copy(zb, acc.at[pl.ds(sid * rows_per_tile, rows_per_tile)])
        plsc.subcore_barrier()

        @pl.loop(0, _NCH)
        def _chunk(ch):
            @pl.loop(wid, _PGRP, step=_NW)
            def _grp(g):
                pltpu.sync_copy(b8.at[pl.ds(pl.multiple_of(
                    ch * (_NR // 128) + g * 8, 8), 8)], bidx)
                pltpu.sync_copy(xt.at[pl.ds(pl.multiple_of(
                    ch * _NR + g * 1024, 1024), 1024)], xb)
                for j in range(8):
                    pltpu.sync_copy(xb.at[pl.ds(j * 128, 128)],
                                    acc.at[bidx.at[j]], add=True)

        plsc.subcore_barrier()
        pltpu.sync_copy(acc.at[pl.ds(sid * rows_per_tile, rows_per_tile)],
                        pparts.at[cid, pl.ds(sid * rows_per_tile,
                                             rows_per_tile)])

    return pl.kernel(
        body,
        out_type=jax.ShapeDtypeStruct((_NC, arows, _CH), jnp.float32),
        mesh=_mesh(),
        compiler_params=pltpu.CompilerParams(use_tc_tiling_on_sc=False),
        scratch_types=[
            pltpu.VMEM_SHARED((arows, _CH), jnp.float32),
            pltpu.VMEM((8, 128), jnp.int32),
            pltpu.VMEM((1024, _CH), jnp.float32),
            pltpu.VMEM((rows_per_tile, _CH), jnp.float32),
            pltpu.SemaphoreType.DMA,
        ],
    )


def _edge1_body(ea, w1t, b1, w2t, b2, out):
    h = jnp.maximum(jnp.dot(ea[...], w1t[...],
                            preferred_element_type=jnp.float32) + b1[...], 0.0)
    out[...] = jnp.dot(h, w2t[...], preferred_element_type=jnp.float32) + b2[...]


def _edge1_call(ea, w1t, b1, w2t, b2):
    nb = _EPAD // _BE
    return pl.pallas_call(
        _edge1_body,
        grid=(nb,),
        in_specs=[
            pl.BlockSpec((_BE, _CH), lambda i: (i, 0)),
            pl.BlockSpec((_CH, _CH), lambda i: (0, 0)),
            pl.BlockSpec((1, _CH), lambda i: (0, 0)),
            pl.BlockSpec((_CH, _CH), lambda i: (0, 0)),
            pl.BlockSpec((1, _CH), lambda i: (0, 0)),
        ],
        out_specs=pl.BlockSpec((_BE, _CH), lambda i: (i, 0)),
        out_shape=jax.ShapeDtypeStruct((_EPAD, _CH), jnp.float32),
    )(ea, w1t, b1, w2t, b2)


def _edge23_body(ea, w1a, b1a, w2a, b2a, w1b, b1b, w2b, b2b, oa, ob):
    x = ea[...]
    for (w1, b1, w2, b2, o) in ((w1a, b1a, w2a, b2a, oa),
                                (w1b, b1b, w2b, b2b, ob)):
        h = jnp.maximum(jnp.dot(x, w1[...],
                                preferred_element_type=jnp.float32) + b1[...],
                        0.0)
        e = jnp.dot(h, w2[...], preferred_element_type=jnp.float32) + b2[...]
        for c in range(_NCH):
            o[c] = e[:, c * _CH:(c + 1) * _CH]


def _edge23_call(ea, wsa, wsb):
    nb = _EPAD // _BE
    wspec = [
        pl.BlockSpec((_CH, _DIM), lambda i: (0, 0)),
        pl.BlockSpec((1, _DIM), lambda i: (0, 0)),
        pl.BlockSpec((_DIM, _DIM), lambda i: (0, 0)),
        pl.BlockSpec((1, _DIM), lambda i: (0, 0)),
    ]
    out_sds = jax.ShapeDtypeStruct((_NCH, _EPAD, _CH), jnp.float32)
    out_spec = pl.BlockSpec((_NCH, _BE, _CH), lambda i: (0, i, 0))
    return pl.pallas_call(
        _edge23_body,
        grid=(nb,),
        in_specs=[pl.BlockSpec((_BE, _CH), lambda i: (i, 0))] + wspec + wspec,
        out_specs=[out_spec, out_spec],
        out_shape=[out_sds, out_sds],
    )(ea, *wsa, *wsb)


def _node_body(nchunk, kin, kmid, xt, parts, w1t, b1, w2t, b2, epsp1, hpre,
               stats):
    del kin, kmid
    i = pl.program_id(0)
    nb = pl.num_programs(0)
    if nchunk == 1:
        x = xt[...]
        agg = parts[0, 0] + parts[1, 0]
    else:
        x = jnp.concatenate([xt[c] for c in range(nchunk)], axis=-1)
        agg = jnp.concatenate(
            [parts[0, c] + parts[1, c] for c in range(nchunk)], axis=-1)
    h0 = epsp1[0, 0] * x + agg
    h1 = jnp.maximum(jnp.dot(h0, w1t[...],
                             preferred_element_type=jnp.float32) + b1[...], 0.0)
    y = jnp.maximum(jnp.dot(h1, w2t[...],
                            preferred_element_type=jnp.float32) + b2[...], 0.0)
    hpre[...] = y

    @pl.when(i == 0)
    def _init():
        c = jnp.mean(y, axis=0, keepdims=True)
        stats[2:3, :] = c
        yc = y - c
        stats[0:1, :] = jnp.sum(yc, axis=0, keepdims=True)
        stats[1:2, :] = jnp.sum(yc * yc, axis=0, keepdims=True)

    @pl.when(i > 0)
    def _accum():
        c = stats[2:3, :]
        rowid = i * _BN + lax.broadcasted_iota(jnp.int32, (_BN, 1), 0)
        yc = jnp.where(rowid < _N, y - c, 0.0)
        stats[0:1, :] = stats[0:1, :] + jnp.sum(yc, axis=0, keepdims=True)
        stats[1:2, :] = stats[1:2, :] + jnp.sum(yc * yc, axis=0,
                                                keepdims=True)
    del nb


def _node_call(nchunk, kin, kmid, xt, parts, w1t, b1, w2t, b2, epsp1):
    if nchunk == 1:
        xt_spec = pl.BlockSpec((_BN, _CH), lambda i: (i, 0))
        xt_arg = xt
    else:
        xt_spec = pl.BlockSpec((nchunk, _BN, _CH), lambda i: (0, i, 0))
        xt_arg = xt.reshape(nchunk, _NR, _CH)
    return pl.pallas_call(
        functools.partial(_node_body, nchunk, kin, kmid),
        grid=(_NBLK,),
        in_specs=[
            xt_spec,
            pl.BlockSpec((_NC, nchunk, _BN, _CH), lambda i: (0, 0, i, 0)),
            pl.BlockSpec((kin, kmid), lambda i: (0, 0)),
            pl.BlockSpec((1, kmid), lambda i: (0, 0)),
            pl.BlockSpec((kmid, _DIM), lambda i: (0, 0)),
            pl.BlockSpec((1, _DIM), lambda i: (0, 0)),
            pl.BlockSpec((1, 1), lambda i: (0, 0), memory_space=pltpu.SMEM),
        ],
        out_specs=[
            pl.BlockSpec((_BN, _DIM), lambda i: (i, 0)),
            pl.BlockSpec((3, _DIM), lambda i: (0, 0)),
        ],
        out_shape=[
            jax.ShapeDtypeStruct((_NR, _DIM), jnp.float32),
            jax.ShapeDtypeStruct((3, _DIM), jnp.float32),
        ],
    )(xt_arg, parts.reshape(_NC, nchunk, _NR, _CH), w1t, b1, w2t, b2, epsp1)


def _bn_body(hpre, stats, g, b, out):
    i = pl.program_id(0)
    d = stats[0:1, :] / _N
    mean = stats[2:3, :] + d
    var = stats[1:2, :] / _N - d * d
    y = (hpre[...] - mean) * lax.rsqrt(var + 1e-5) * g[...] + b[...]
    rowid = i * _BN + lax.broadcasted_iota(jnp.int32, (_BN, 1), 0)
    y = jnp.where(rowid < _N, y, 0.0)
    for c in range(_NCH):
        out[c] = y[:, c * _CH:(c + 1) * _CH]


def _bn_call(hpre, stats, g, b):
    return pl.pallas_call(
        _bn_body,
        grid=(_NBLK,),
        in_specs=[
            pl.BlockSpec((_BN, _DIM), lambda i: (i, 0)),
            pl.BlockSpec((3, _DIM), lambda i: (0, 0)),
            pl.BlockSpec((1, _DIM), lambda i: (0, 0)),
            pl.BlockSpec((1, _DIM), lambda i: (0, 0)),
        ],
        out_specs=pl.BlockSpec((_NCH, _BN, _CH), lambda i: (0, i, 0)),
        out_shape=jax.ShapeDtypeStruct((_NCH, _NR, _CH), jnp.float32),
    )(hpre, stats, g, b)


def _fc_body(pp1, pp2, pp3, pp4, b2d, w1t, b1, w2t, b2, w3t, b3, w4t, b4, out):
    iota_g = lax.broadcasted_iota(jnp.int32, (_G, _BN), 0)

    def cnt_step(j, c):
        blk = b2d[pl.ds(j, 1), :]
        return c + jnp.sum((blk == iota_g).astype(jnp.float32), axis=1,
                           keepdims=True)

    cnt = lax.fori_loop(0, _NBLK, cnt_step,
                        jnp.zeros((_G, 1), jnp.float32))
    denom = jnp.maximum(cnt, 1.0)
    cols = []
    for pp in (pp1, pp2, pp3, pp4):
        for c in range(_NCH):
            cols.append(pp[0, c * _GS:c * _GS + _G, :]
                        + pp[1, c * _GS:c * _GS + _G, :])
    pooled = jnp.concatenate(cols, axis=-1) / denom
    h = jnp.maximum(jnp.dot(pooled, w1t[...],
                            preferred_element_type=jnp.float32) + b1[...], 0.0)
    h = jnp.maximum(jnp.dot(h, w2t[...],
                            preferred_element_type=jnp.float32) + b2[...], 0.0)
    h = jnp.maximum(jnp.dot(h, w3t[...],
                            preferred_element_type=jnp.float32) + b3[...], 0.0)
    out[...] = jnp.dot(h, w4t[...],
                       preferred_element_type=jnp.float32) + b4[...]


def _fc_call(pps, b2d, fcw):
    arows = _NCH * _GS
    pspec = pl.BlockSpec((_NC, arows, _CH), lambda: (0, 0, 0))
    return pl.pallas_call(
        _fc_body,
        grid=(),
        in_specs=[pspec, pspec, pspec, pspec,
                  pl.BlockSpec((_NBLK, _BN), lambda: (0, 0)),
                  pl.BlockSpec((4 * _DIM, _DIM), lambda: (0, 0)),
                  pl.BlockSpec((1, _DIM), lambda: (0, 0)),
                  pl.BlockSpec((_DIM, _DIM), lambda: (0, 0)),
                  pl.BlockSpec((1, _DIM), lambda: (0, 0)),
                  pl.BlockSpec((_DIM, _DIM), lambda: (0, 0)),
                  pl.BlockSpec((1, _DIM), lambda: (0, 0)),
                  pl.BlockSpec((_DIM, 128), lambda: (0, 0)),
                  pl.BlockSpec((1, 128), lambda: (0, 0))],
        out_specs=pl.BlockSpec((_G, 128), lambda: (0, 0)),
        out_shape=jax.ShapeDtypeStruct((_G, 128), jnp.float32),
    )(*pps, b2d, *fcw)


def _pad2(w, r, c):
    return jnp.pad(w, ((0, r - w.shape[0]), (0, c - w.shape[1])))


def _prep_edge_w(p, d1):
    w1t = _pad2(p['be_w1'].T, _CH, d1)
    b1 = _pad2(p['be_b1'][None, :], 1, d1)
    w2t = _pad2(p['be_w2'].T, d1, d1)
    b2 = _pad2(p['be_b2'][None, :], 1, d1)
    return w1t, b1, w2t, b2


def kernel(x, edge_index, edge_attr, batch, params):
    p = params
    src = edge_index[0]
    dst = edge_index[1]
    pe = _EPAD - _E
    pad_src = _N + (jnp.arange(pe, dtype=jnp.int32) % 64)
    pad_dst = _N + (jnp.arange(pe, dtype=jnp.int32) % 128)
    src_p = jnp.concatenate([src, pad_src])
    dst_p = jnp.concatenate([dst, pad_dst])
    src1 = src_p.reshape(_EPAD // 128, 128)
    offs = (jnp.arange(_NCH, dtype=jnp.int32) * _NR)[:, None]
    src8 = (src_p[None, :] + offs).reshape(_NCH * _EPAD // 128, 128)
    dstr = dst_p.reshape(_EPAD // 128, 128)

    batch_p = jnp.concatenate(
        [batch, _G + (jnp.arange(_NR - _N, dtype=jnp.int32) % 16)])
    goffs = (jnp.arange(_NCH, dtype=jnp.int32) * _GS)[:, None]
    b8 = (batch_p[None, :] + goffs).reshape(_NCH * _NR // 128, 128)
    b2d = batch_p.reshape(_NBLK, _BN)

    ea = jnp.pad(edge_attr, ((0, pe), (0, _CH - _NF)))
    xt0 = jnp.pad(x, ((0, _NR - _N), (0, _CH - _NF)))

    e1 = _edge1_call(ea, *_prep_edge_w(p['conv1'], _CH))
    e2c, e3c = _edge23_call(ea, _prep_edge_w(p['conv2'], _DIM),
                            _prep_edge_w(p['conv3'], _DIM))

    scat1 = _make_scatter(1)
    scat8 = _make_scatter(_NCH)
    pool = _make_pool()

    def conv_w(cp, kin, kmid):
        w1t = _pad2(cp['mlp_w1'].T, kin, kmid)
        b1 = _pad2(cp['mlp_b1'][None, :], 1, kmid)
        w2t = _pad2(cp['mlp_w2'].T, kmid, _DIM)
        b2 = cp['mlp_b2'][None, :]
        epsp1 = (1.0 + cp['eps']).reshape(1, 1)
        return w1t, b1, w2t, b2, epsp1

    parts1 = scat1(xt0, e1.reshape(_EPAD, _CH), src1, dstr)
    hp1, st1 = _node_call(1, _CH, _CH, xt0, parts1,
                          *conv_w(p['conv1'], _CH, _CH))
    xt1 = _bn_call(hp1, st1, p['bn1_g'][None, :], p['bn1_b'][None, :])
    xt1f = xt1.reshape(_NCH * _NR, _CH)
    pp1 = pool(xt1f, b8)

    def layer(xtf, ec, cp, g, b):
        parts = scat8(xtf, ec.reshape(_NCH * _EPAD, _CH), src8, dstr)
        hp, st = _node_call(_NCH, _DIM, _DIM, xtf, parts,
                            *conv_w(cp, _DIM, _DIM))
        xt_n = _bn_call(hp, st, g[None, :], b[None, :])
        xtnf = xt_n.reshape(_NCH * _NR, _CH)
        return xtnf, pool(xtnf, b8)

    xt2f, pp2 = layer(xt1f, e2c, p['conv2'], p['bn2_g'], p['bn2_b'])
    xt3f, pp3 = layer(xt2f, e3c, p['conv3'], p['bn3_g'], p['bn3_b'])
    _, pp4 = layer(xt3f, e3c, p['conv3'], p['bn4_g'], p['bn4_b'])

    fcw = (p['fc1_w'].T, p['fc1_b'][None, :],
           p['fc2_w'].T, p['fc2_b'][None, :],
           p['fc3_w'].T, p['fc3_b'][None, :],
           _pad2(p['fc4_w'].T, _DIM, 128), _pad2(p['fc4_b'][None, :], 1, 128))
    out = _fc_call([pp1, pp2, pp3, pp4], b2d, fcw)
    return out[:, 0]


# speedup vs baseline: 1.1517x; 1.1517x over previous
import functools

import jax
import jax.numpy as jnp
from jax import lax
from jax.experimental import pallas as pl
from jax.experimental.pallas import tpu as pltpu
from jax.experimental.pallas import tpu_sc as plsc

_N = 50000
_E = 800000
_G = 256
_NF = 21
_DIM = 256

_NC = 2
_NS = 16
_NW = _NC * _NS

_CH = 32
_NCH = _DIM // _CH
_NR = 50176
_W = 1024
_KI = _W // 128
_NWIN = 25
_EW = _NWIN * _W
_EPAD = _NW * _EW
_GS = 272
_PGRP = _NR // 1024
_BN = 512
_NBLK = _NR // _BN
_BE = 2048


def _mesh():
    return plsc.VectorSubcoreMesh(core_axis_name="c", subcore_axis_name="s",
                                  num_cores=_NC, num_subcores=_NS)


def _make_scatter(nchunk):
    acc_rows_per_tile = _NR // _NS
    n_zero = acc_rows_per_tile // 64
    _Q = _W // 4

    def body(xt, e, srcn, dstr, parts, acc, sidx, didx, xga, xgb, eb, sema,
             semb):
        cid = lax.axis_index("c")
        sid = lax.axis_index("s")
        wid = sid * _NC + cid
        bufs = (xga, xgb)
        sems = (sema, semb)

        @pl.loop(0, nchunk)
        def _chunk(ch):
            @pl.loop(0, 64)
            def _fill_z(r):
                z = jnp.zeros((16,), jnp.float32)
                xga[r, 0:16] = z
                xga[r, 16:32] = z

            @pl.loop(0, n_zero)
            def _zero(i):
                pltpu.sync_copy(
                    xga.at[pl.ds(0, 64)],
                    acc.at[pl.ds(pl.multiple_of(
                        sid * acc_rows_per_tile + i * 64, 64), 64)])

            plsc.subcore_barrier()

            @pl.loop(0, _NWIN)
            def _win(w):
                base = pl.multiple_of(wid * _EW + w * _W, _W)
                r0 = pl.multiple_of(ch * (_EPAD // 128) + base // 128, 8)
                pltpu.sync_copy(srcn.at[pl.ds(r0, _KI)], sidx)
                pltpu.sync_copy(dstr.at[pl.ds(pl.multiple_of(base // 128, 8),
                                              _KI)], didx)
                def gather(ch, h, buf, sem):
                    return [pltpu.async_copy(
                        xt.at[sidx.at[h * 2 + j]],
                        buf.at[pl.ds(j * 128, 128)], sem) for j in range(2)]

                descs = {0: gather(ch, 0, bufs[0], sems[0])}
                for h in range(4):
                    p = h % 2
                    xg = bufs[p]
                    for d in descs.pop(p):
                        d.wait()
                    if h < 3:
                        descs[1 - p] = gather(ch, h + 1, bufs[1 - p],
                                              sems[1 - p])
                    pltpu.sync_copy(
                        e.at[pl.ds(pl.multiple_of((base + h * _Q) // 8,
                                                  _Q // 8), _Q // 8),
                             slice(None),
                             pl.ds(ch * _CH, _CH)], eb)

                    @pl.loop(0, _Q // 8, unroll=2)
                    def _msg(o):
                        for q in range(8):
                            i = o * 8 + q
                            a = xg[i, 0:16] + eb[o, q, 0:16]
                            xg[i, 0:16] = jnp.maximum(a, 0.0)
                            b = xg[i, 16:32] + eb[o, q, 16:32]
                            xg[i, 16:32] = jnp.maximum(b, 0.0)

                    for j in range(2):
                        pltpu.sync_copy(xg.at[pl.ds(j * 128, 128)],
                                        acc.at[didx.at[h * 2 + j]],
                                        add=True)

            plsc.subcore_barrier()

            row = pl.multiple_of(sid * acc_rows_per_tile, 64)
            pltpu.sync_copy(
                acc.at[pl.ds(row, acc_rows_per_tile)],
                parts.at[cid, pl.ds(row, acc_rows_per_tile),
                         pl.ds(ch * _CH, _CH)])

            plsc.subcore_barrier()

    return pl.kernel(
        body,
        out_type=jax.ShapeDtypeStruct((_NC, _NR, nchunk * _CH), jnp.float32),
        mesh=_mesh(),
        compiler_params=pltpu.CompilerParams(use_tc_tiling_on_sc=False),
        scratch_types=[
            pltpu.VMEM_SHARED((_NR, _CH), jnp.float32),
            pltpu.VMEM((_KI, 128), jnp.int32),
            pltpu.VMEM((_KI, 128), jnp.int32),
            pltpu.VMEM((_Q, _CH), jnp.float32),
            pltpu.VMEM((_Q, _CH), jnp.float32),
            pltpu.VMEM((_Q // 8, 8, _CH), jnp.float32),
            pltpu.SemaphoreType.DMA,
            pltpu.SemaphoreType.DMA,
        ],
    )


def _make_pool():
    arows = _NCH * _GS
    rows_per_tile = arows // _NS

    def body(xt, b8, pparts, acc, bidx, xb, zb, sem):
        del sem
        cid = lax.axis_index("c")
        sid = lax.axis_index("s")
        wid = sid * _NC + cid

        @pl.loop(0, rows_per_tile)
        def _fill_zb(r):
            z = jnp.zeros((16,), jnp.float32)
            zb[r, 0:16] = z
            zb[r, 16:32] = z

        pltpu.sync_copy(zb, acc.at[pl.ds(sid * rows_per_tile, rows_per_tile)])
        plsc.subcore_barrier()

        @pl.loop(0, _NCH)
        def _chunk(ch):
            @pl.loop(wid, _PGRP, step=_NW)
            def _grp(g):
                pltpu.sync_copy(b8.at[pl.ds(pl.multiple_of(
                    ch * (_NR // 128) + g * 8, 8), 8)], bidx)
                pltpu.sync_copy(xt.at[pl.ds(pl.multiple_of(
                    ch * _NR + g * 1024, 1024), 1024)], xb)
                for j in range(8):
                    pltpu.sync_copy(xb.at[pl.ds(j * 128, 128)],
                                    acc.at[bidx.at[j]], add=True)

        plsc.subcore_barrier()
        pltpu.sync_copy(acc.at[pl.ds(sid * rows_per_tile, rows_per_tile)],
                        pparts.at[cid, pl.ds(sid * rows_per_tile,
                                             rows_per_tile)])

    return pl.kernel(
        body,
        out_type=jax.ShapeDtypeStruct((_NC, arows, _CH), jnp.float32),
        mesh=_mesh(),
        compiler_params=pltpu.CompilerParams(use_tc_tiling_on_sc=False),
        scratch_types=[
            pltpu.VMEM_SHARED((arows, _CH), jnp.float32),
            pltpu.VMEM((8, 128), jnp.int32),
            pltpu.VMEM((1024, _CH), jnp.float32),
            pltpu.VMEM((rows_per_tile, _CH), jnp.float32),
            pltpu.SemaphoreType.DMA,
        ],
    )


def _edge1_body(ea, w1t, b1, w2t, b2, out):
    h = jnp.maximum(jnp.dot(ea[...], w1t[...],
                            preferred_element_type=jnp.float32) + b1[...], 0.0)
    ee = jnp.dot(h, w2t[...], preferred_element_type=jnp.float32) + b2[...]
    out[...] = ee.reshape(_BE // 8, 8, _CH)


def _edge1_call(ea, w1t, b1, w2t, b2):
    nb = _EPAD // _BE
    return pl.pallas_call(
        _edge1_body,
        grid=(nb,),
        in_specs=[
            pl.BlockSpec((_BE, _CH), lambda i: (i, 0)),
            pl.BlockSpec((_CH, _CH), lambda i: (0, 0)),
            pl.BlockSpec((1, _CH), lambda i: (0, 0)),
            pl.BlockSpec((_CH, _CH), lambda i: (0, 0)),
            pl.BlockSpec((1, _CH), lambda i: (0, 0)),
        ],
        out_specs=pl.BlockSpec((_BE // 8, 8, _CH), lambda i: (i, 0, 0)),
        out_shape=jax.ShapeDtypeStruct((_EPAD // 8, 8, _CH), jnp.float32),
    )(ea, w1t, b1, w2t, b2)


def _edge23_body(ea, w1a, b1a, w2a, b2a, w1b, b1b, w2b, b2b, oa, ob):
    x = ea[...]
    for (w1, b1, w2, b2, o) in ((w1a, b1a, w2a, b2a, oa),
                                (w1b, b1b, w2b, b2b, ob)):
        h = jnp.maximum(jnp.dot(x, w1[...],
                                preferred_element_type=jnp.float32) + b1[...],
                        0.0)
        ee = jnp.dot(h, w2[...], preferred_element_type=jnp.float32) + b2[...]
        o[...] = ee.reshape(_BE // 8, 8, _DIM)


def _edge23_call(ea, wsa, wsb):
    nb = _EPAD // _BE
    wspec = [
        pl.BlockSpec((_CH, _DIM), lambda i: (0, 0)),
        pl.BlockSpec((1, _DIM), lambda i: (0, 0)),
        pl.BlockSpec((_DIM, _DIM), lambda i: (0, 0)),
        pl.BlockSpec((1, _DIM), lambda i: (0, 0)),
    ]
    out_sds = jax.ShapeDtypeStruct((_EPAD // 8, 8, _DIM), jnp.float32)
    out_spec = pl.BlockSpec((_BE // 8, 8, _DIM), lambda i: (i, 0, 0))
    return pl.pallas_call(
        _edge23_body,
        grid=(nb,),
        in_specs=[pl.BlockSpec((_BE, _CH), lambda i: (i, 0))] + wspec + wspec,
        out_specs=[out_spec, out_spec],
        out_shape=[out_sds, out_sds],
    )(ea, *wsa, *wsb)


def _node_body(nchunk, kin, kmid, xt, parts, w1t, b1, w2t, b2, epsp1, hpre,
               stats):
    del kin, kmid
    i = pl.program_id(0)
    nb = pl.num_programs(0)
    x = xt[...]
    agg = parts[0] + parts[1]
    h0 = epsp1[0, 0] * x + agg
    h1 = jnp.maximum(jnp.dot(h0, w1t[...],
                             preferred_element_type=jnp.float32) + b1[...], 0.0)
    y = jnp.maximum(jnp.dot(h1, w2t[...],
                            preferred_element_type=jnp.float32) + b2[...], 0.0)
    hpre[...] = y

    @pl.when(i == 0)
    def _init():
        c = jnp.mean(y, axis=0, keepdims=True)
        stats[2:3, :] = c
        yc = y - c
        stats[0:1, :] = jnp.sum(yc, axis=0, keepdims=True)
        stats[1:2, :] = jnp.sum(yc * yc, axis=0, keepdims=True)

    @pl.when(i > 0)
    def _accum():
        c = stats[2:3, :]
        rowid = i * _BN + lax.broadcasted_iota(jnp.int32, (_BN, 1), 0)
        yc = jnp.where(rowid < _N, y - c, 0.0)
        stats[0:1, :] = stats[0:1, :] + jnp.sum(yc, axis=0, keepdims=True)
        stats[1:2, :] = stats[1:2, :] + jnp.sum(yc * yc, axis=0,
                                                keepdims=True)
    del nb


def _node_call(nchunk, kin, kmid, xt, parts, w1t, b1, w2t, b2, epsp1):
    xt_spec = pl.BlockSpec((_BN, nchunk * _CH), lambda i: (i, 0))
    return pl.pallas_call(
        functools.partial(_node_body, nchunk, kin, kmid),
        grid=(_NBLK,),
        in_specs=[
            xt_spec,
            pl.BlockSpec((_NC, _BN, nchunk * _CH), lambda i: (0, i, 0)),
            pl.BlockSpec((kin, kmid), lambda i: (0, 0)),
            pl.BlockSpec((1, kmid), lambda i: (0, 0)),
            pl.BlockSpec((kmid, _DIM), lambda i: (0, 0)),
            pl.BlockSpec((1, _DIM), lambda i: (0, 0)),
            pl.BlockSpec((1, 1), lambda i: (0, 0), memory_space=pltpu.SMEM),
        ],
        out_specs=[
            pl.BlockSpec((_BN, _DIM), lambda i: (i, 0)),
            pl.BlockSpec((3, _DIM), lambda i: (0, 0)),
        ],
        out_shape=[
            jax.ShapeDtypeStruct((_NR, _DIM), jnp.float32),
            jax.ShapeDtypeStruct((3, _DIM), jnp.float32),
        ],
    )(xt, parts, w1t, b1, w2t, b2, epsp1)


def _bn_body(hpre, stats, g, b, out):
    i = pl.program_id(0)
    d = stats[0:1, :] / _N
    mean = stats[2:3, :] + d
    var = stats[1:2, :] / _N - d * d
    y = (hpre[...] - mean) * lax.rsqrt(var + 1e-5) * g[...] + b[...]
    rowid = i * _BN + lax.broadcasted_iota(jnp.int32, (_BN, 1), 0)
    out[...] = jnp.where(rowid < _N, y, 0.0)


def _bn_call(hpre, stats, g, b):
    return pl.pallas_call(
        _bn_body,
        grid=(_NBLK,),
        in_specs=[
            pl.BlockSpec((_BN, _DIM), lambda i: (i, 0)),
            pl.BlockSpec((3, _DIM), lambda i: (0, 0)),
            pl.BlockSpec((1, _DIM), lambda i: (0, 0)),
            pl.BlockSpec((1, _DIM), lambda i: (0, 0)),
        ],
        out_specs=pl.BlockSpec((_BN, _DIM), lambda i: (i, 0)),
        out_shape=jax.ShapeDtypeStruct((_NR, _DIM), jnp.float32),
    )(hpre, stats, g, b)


def _fc_body(pp1, pp2, pp3, pp4, b2d, w1t, b1, w2t, b2, w3t, b3, w4t, b4, out):
    iota_g = lax.broadcasted_iota(jnp.int32, (_G, _BN), 0)

    def cnt_step(j, c):
        blk = b2d[pl.ds(j, 1), :]
        return c + jnp.sum((blk == iota_g).astype(jnp.float32), axis=1,
                           keepdims=True)

    cnt = lax.fori_loop(0, _NBLK, cnt_step,
                        jnp.zeros((_G, 1), jnp.float32))
    denom = jnp.maximum(cnt, 1.0)
    cols = []
    for pp in (pp1, pp2, pp3, pp4):
        for c in range(_NCH):
            cols.append(pp[0, c * _GS:c * _GS + _G, :]
                        + pp[1, c * _GS:c * _GS + _G, :])
    pooled = jnp.concatenate(cols, axis=-1) / denom
    h = jnp.maximum(jnp.dot(pooled, w1t[...],
                            preferred_element_type=jnp.float32) + b1[...], 0.0)
    h = jnp.maximum(jnp.dot(h, w2t[...],
                            preferred_element_type=jnp.float32) + b2[...], 0.0)
    h = jnp.maximum(jnp.dot(h, w3t[...],
                            preferred_element_type=jnp.float32) + b3[...], 0.0)
    out[...] = jnp.dot(h, w4t[...],
                       preferred_element_type=jnp.float32) + b4[...]


def _fc_call(pps, b2d, fcw):
    arows = _NCH * _GS
    pspec = pl.BlockSpec((_NC, arows, _CH), lambda: (0, 0, 0))
    return pl.pallas_call(
        _fc_body,
        grid=(),
        in_specs=[pspec, pspec, pspec, pspec,
                  pl.BlockSpec((_NBLK, _BN), lambda: (0, 0)),
                  pl.BlockSpec((4 * _DIM, _DIM), lambda: (0, 0)),
                  pl.BlockSpec((1, _DIM), lambda: (0, 0)),
                  pl.BlockSpec((_DIM, _DIM), lambda: (0, 0)),
                  pl.BlockSpec((1, _DIM), lambda: (0, 0)),
                  pl.BlockSpec((_DIM, _DIM), lambda: (0, 0)),
                  pl.BlockSpec((1, _DIM), lambda: (0, 0)),
                  pl.BlockSpec((_DIM, 128), lambda: (0, 0)),
                  pl.BlockSpec((1, 128), lambda: (0, 0))],
        out_specs=pl.BlockSpec((_G, 128), lambda: (0, 0)),
        out_shape=jax.ShapeDtypeStruct((_G, 128), jnp.float32),
    )(*pps, b2d, *fcw)


def _pad2(w, r, c):
    return jnp.pad(w, ((0, r - w.shape[0]), (0, c - w.shape[1])))


def _prep_edge_w(p, d1):
    w1t = _pad2(p['be_w1'].T, _CH, d1)
    b1 = _pad2(p['be_b1'][None, :], 1, d1)
    w2t = _pad2(p['be_w2'].T, d1, d1)
    b2 = _pad2(p['be_b2'][None, :], 1, d1)
    return w1t, b1, w2t, b2


def kernel(x, edge_index, edge_attr, batch, params):
    p = params
    src = edge_index[0]
    dst = edge_index[1]
    pe = _EPAD - _E
    pad_src = _N + (jnp.arange(pe, dtype=jnp.int32) % 64)
    pad_dst = _N + (jnp.arange(pe, dtype=jnp.int32) % 128)
    src_p = jnp.concatenate([src, pad_src])
    dst_p = jnp.concatenate([dst, pad_dst])
    src1 = src_p.reshape(_EPAD // 128, 128)
    offs = (jnp.arange(_NCH, dtype=jnp.int32) * _NR)[:, None]
    src8 = (src_p[None, :] + offs).reshape(_NCH * _EPAD // 128, 128)
    dstr = dst_p.reshape(_EPAD // 128, 128)

    batch_p = jnp.concatenate(
        [batch, _G + (jnp.arange(_NR - _N, dtype=jnp.int32) % 16)])
    goffs = (jnp.arange(_NCH, dtype=jnp.int32) * _GS)[:, None]
    b8 = (batch_p[None, :] + goffs).reshape(_NCH * _NR // 128, 128)
    b2d = batch_p.reshape(_NBLK, _BN)

    ea = jnp.pad(edge_attr, ((0, pe), (0, _CH - _NF)))
    xt0 = jnp.pad(x, ((0, _NR - _N), (0, _CH - _NF)))

    e1 = _edge1_call(ea, *_prep_edge_w(p['conv1'], _CH))
    e2c, e3c = _edge23_call(ea, _prep_edge_w(p['conv2'], _DIM),
                            _prep_edge_w(p['conv3'], _DIM))

    scat1 = _make_scatter(1)
    scat8 = _make_scatter(_NCH)
    pool = _make_pool()

    def conv_w(cp, kin, kmid):
        w1t = _pad2(cp['mlp_w1'].T, kin, kmid)
        b1 = _pad2(cp['mlp_b1'][None, :], 1, kmid)
        w2t = _pad2(cp['mlp_w2'].T, kmid, _DIM)
        b2 = cp['mlp_b2'][None, :]
        epsp1 = (1.0 + cp['eps']).reshape(1, 1)
        return w1t, b1, w2t, b2, epsp1

    parts1 = scat1(xt0, e1, src1, dstr)
    hp1, st1 = _node_call(1, _CH, _CH, xt0, parts1,
                          *conv_w(p['conv1'], _CH, _CH))
    xt1 = _bn_call(hp1, st1, p['bn1_g'][None, :], p['bn1_b'][None, :])

    def packed(xt_l):
        return jnp.transpose(xt_l.reshape(_NR, _NCH, _CH),
                             (1, 0, 2)).reshape(_NCH * _NR, _CH)

    xt1p = packed(xt1)
    pp1 = pool(xt1p, b8)

    def layer(xt_l, xt_lp, ec, cp, g, b):
        parts = scat8(xt_lp, ec, src8, dstr)
        hp, st = _node_call(_NCH, _DIM, _DIM, xt_l, parts,
                            *conv_w(cp, _DIM, _DIM))
        xt_n = _bn_call(hp, st, g[None, :], b[None, :])
        xt_np = packed(xt_n)
        return xt_n, xt_np, pool(xt_np, b8)

    xt2, xt2p, pp2 = layer(xt1, xt1p, e2c, p['conv2'], p['bn2_g'], p['bn2_b'])
    xt3, xt3p, pp3 = layer(xt2, xt2p, e3c, p['conv3'], p['bn3_g'], p['bn3_b'])
    _, _, pp4 = layer(xt3, xt3p, e3c, p['conv3'], p['bn4_g'], p['bn4_b'])

    fcw = (p['fc1_w'].T, p['fc1_b'][None, :],
           p['fc2_w'].T, p['fc2_b'][None, :],
           p['fc3_w'].T, p['fc3_b'][None, :],
           _pad2(p['fc4_w'].T, _DIM, 128), _pad2(p['fc4_b'][None, :], 1, 128))
    out = _fc_call([pp1, pp2, pp3, pp4], b2d, fcw)
    return out[:, 0]


# speedup vs baseline: 1.2028x; 1.0443x over previous
import functools

import jax
import jax.numpy as jnp
from jax import lax
from jax.experimental import pallas as pl
from jax.experimental.pallas import tpu as pltpu
from jax.experimental.pallas import tpu_sc as plsc

_N = 50000
_E = 800000
_G = 256
_NF = 21
_DIM = 256

_NC = 2
_NS = 16
_NW = _NC * _NS

_CH = 32
_NCH = _DIM // _CH
_NR = 50176
_W = 1024
_KI = _W // 128
_NWIN = 25
_EW = _NWIN * _W
_EPAD = _NW * _EW
_GS = 272
_PGRP = _NR // 1024
_BN = 512
_NBLK = _NR // _BN
_BE = 2048


def _mesh():
    return plsc.VectorSubcoreMesh(core_axis_name="c", subcore_axis_name="s",
                                  num_cores=_NC, num_subcores=_NS)


def _make_scatter(nchunk):
    acc_rows_per_tile = _NR // _NS
    n_zero = acc_rows_per_tile // 64
    _Q = _W // 4

    def body(xt, e, srcn, dstr, parts, acc, sidx, didx, xga, xgb, xgc, eb,
             sga, sgb, sgc, ssa, ssb, ssc):
        cid = lax.axis_index("c")
        sid = lax.axis_index("s")
        wid = sid * _NC + cid
        bufs = (xga, xgb, xgc)
        gsems = (sga, sgb, sgc)
        ssems = (ssa, ssb, ssc)

        @pl.loop(0, nchunk)
        def _chunk(ch):
            @pl.loop(0, 64)
            def _fill_z(r):
                z = jnp.zeros((16,), jnp.float32)
                xga[r, 0:16] = z
                xga[r, 16:32] = z

            @pl.loop(0, n_zero)
            def _zero(i):
                pltpu.sync_copy(
                    xga.at[pl.ds(0, 64)],
                    acc.at[pl.ds(pl.multiple_of(
                        sid * acc_rows_per_tile + i * 64, 64), 64)])

            plsc.subcore_barrier()

            @pl.loop(0, _NWIN)
            def _win(w):
                base = pl.multiple_of(wid * _EW + w * _W, _W)
                r0 = pl.multiple_of(ch * (_EPAD // 128) + base // 128, 8)
                pltpu.sync_copy(srcn.at[pl.ds(r0, _KI)], sidx)
                pltpu.sync_copy(dstr.at[pl.ds(pl.multiple_of(base // 128, 8),
                                              _KI)], didx)

                def gather(u):
                    b = u % 3
                    return pltpu.async_copy(
                        xt.at[sidx.at[u]], bufs[b], gsems[b])

                gd = {0: gather(0)}
                sd = {}
                for u in range(8):
                    b = u % 3
                    gd.pop(u).wait()
                    if u < 7:
                        nb = (u + 1) % 3
                        if nb in sd:
                            sd.pop(nb).wait()
                        gd[u + 1] = gather(u + 1)
                    if u % 2 == 0:
                        pltpu.sync_copy(
                            e.at[pl.ds(pl.multiple_of(
                                (base + u * 128) // 8, 32), 32),
                                 slice(None),
                                 pl.ds(ch * _CH, _CH)], eb)
                    xg = bufs[b]
                    eo = 16 * (u % 2)

                    @pl.loop(0, 16, unroll=4)
                    def _msg(o):
                        for q in range(8):
                            i = o * 8 + q
                            a = xg[i, 0:16] + eb[eo + o, q, 0:16]
                            xg[i, 0:16] = jnp.maximum(a, 0.0)
                            bb = xg[i, 16:32] + eb[eo + o, q, 16:32]
                            xg[i, 16:32] = jnp.maximum(bb, 0.0)

                    sd[b] = pltpu.async_copy(xg, acc.at[didx.at[u]],
                                             ssems[b], add=True)
                for d in sd.values():
                    d.wait()

            plsc.subcore_barrier()

            row = pl.multiple_of(sid * acc_rows_per_tile, 64)
            pltpu.sync_copy(
                acc.at[pl.ds(row, acc_rows_per_tile)],
                parts.at[cid, pl.ds(row, acc_rows_per_tile),
                         pl.ds(ch * _CH, _CH)])

            plsc.subcore_barrier()

    return pl.kernel(
        body,
        out_type=jax.ShapeDtypeStruct((_NC, _NR, nchunk * _CH), jnp.float32),
        mesh=_mesh(),
        compiler_params=pltpu.CompilerParams(use_tc_tiling_on_sc=False),
        scratch_types=[
            pltpu.VMEM_SHARED((_NR, _CH), jnp.float32),
            pltpu.VMEM((_KI, 128), jnp.int32),
            pltpu.VMEM((_KI, 128), jnp.int32),
            pltpu.VMEM((128, _CH), jnp.float32),
            pltpu.VMEM((128, _CH), jnp.float32),
            pltpu.VMEM((128, _CH), jnp.float32),
            pltpu.VMEM((_Q // 8, 8, _CH), jnp.float32),
            pltpu.SemaphoreType.DMA,
            pltpu.SemaphoreType.DMA,
            pltpu.SemaphoreType.DMA,
            pltpu.SemaphoreType.DMA,
            pltpu.SemaphoreType.DMA,
            pltpu.SemaphoreType.DMA,
        ],
    )


def _make_pool():
    arows = _NCH * _GS
    rows_per_tile = arows // _NS

    def body(xt, b8, pparts, acc, bidx, xb, zb, sem):
        del sem
        cid = lax.axis_index("c")
        sid = lax.axis_index("s")
        wid = sid * _NC + cid

        @pl.loop(0, rows_per_tile)
        def _fill_zb(r):
            z = jnp.zeros((16,), jnp.float32)
            zb[r, 0:16] = z
            zb[r, 16:32] = z

        pltpu.sync_copy(zb, acc.at[pl.ds(sid * rows_per_tile, rows_per_tile)])
        plsc.subcore_barrier()

        @pl.loop(0, _NCH)
        def _chunk(ch):
            @pl.loop(wid, _PGRP, step=_NW)
            def _grp(g):
                pltpu.sync_copy(b8.at[pl.ds(pl.multiple_of(
                    ch * (_NR // 128) + g * 8, 8), 8)], bidx)
                pltpu.sync_copy(xt.at[pl.ds(pl.multiple_of(
                    ch * _NR + g * 1024, 1024), 1024)], xb)
                for j in range(8):
                    pltpu.sync_copy(xb.at[pl.ds(j * 128, 128)],
                                    acc.at[bidx.at[j]], add=True)

        plsc.subcore_barrier()
        pltpu.sync_copy(acc.at[pl.ds(sid * rows_per_tile, rows_per_tile)],
                        pparts.at[cid, pl.ds(sid * rows_per_tile,
                                             rows_per_tile)])

    return pl.kernel(
        body,
        out_type=jax.ShapeDtypeStruct((_NC, arows, _CH), jnp.float32),
        mesh=_mesh(),
        compiler_params=pltpu.CompilerParams(use_tc_tiling_on_sc=False),
        scratch_types=[
            pltpu.VMEM_SHARED((arows, _CH), jnp.float32),
            pltpu.VMEM((8, 128), jnp.int32),
            pltpu.VMEM((1024, _CH), jnp.float32),
            pltpu.VMEM((rows_per_tile, _CH), jnp.float32),
            pltpu.SemaphoreType.DMA,
        ],
    )


def _edge1_body(ea, w1t, b1, w2t, b2, out):
    h = jnp.maximum(jnp.dot(ea[...], w1t[...],
                            preferred_element_type=jnp.float32) + b1[...], 0.0)
    ee = jnp.dot(h, w2t[...], preferred_element_type=jnp.float32) + b2[...]
    out[...] = ee.reshape(_BE // 8, 8, _CH)


def _edge1_call(ea, w1t, b1, w2t, b2):
    nb = _EPAD // _BE
    return pl.pallas_call(
        _edge1_body,
        grid=(nb,),
        in_specs=[
            pl.BlockSpec((_BE, _CH), lambda i: (i, 0)),
            pl.BlockSpec((_CH, _CH), lambda i: (0, 0)),
            pl.BlockSpec((1, _CH), lambda i: (0, 0)),
            pl.BlockSpec((_CH, _CH), lambda i: (0, 0)),
            pl.BlockSpec((1, _CH), lambda i: (0, 0)),
        ],
        out_specs=pl.BlockSpec((_BE // 8, 8, _CH), lambda i: (i, 0, 0)),
        out_shape=jax.ShapeDtypeStruct((_EPAD // 8, 8, _CH), jnp.float32),
    )(ea, w1t, b1, w2t, b2)


def _edge23_body(ea, w1a, b1a, w2a, b2a, w1b, b1b, w2b, b2b, oa, ob):
    x = ea[...]
    for (w1, b1, w2, b2, o) in ((w1a, b1a, w2a, b2a, oa),
                                (w1b, b1b, w2b, b2b, ob)):
        h = jnp.maximum(jnp.dot(x, w1[...],
                                preferred_element_type=jnp.float32) + b1[...],
                        0.0)
        ee = jnp.dot(h, w2[...], preferred_element_type=jnp.float32) + b2[...]
        o[...] = ee.reshape(_BE // 8, 8, _DIM)


def _edge23_call(ea, wsa, wsb):
    nb = _EPAD // _BE
    wspec = [
        pl.BlockSpec((_CH, _DIM), lambda i: (0, 0)),
        pl.BlockSpec((1, _DIM), lambda i: (0, 0)),
        pl.BlockSpec((_DIM, _DIM), lambda i: (0, 0)),
        pl.BlockSpec((1, _DIM), lambda i: (0, 0)),
    ]
    out_sds = jax.ShapeDtypeStruct((_EPAD // 8, 8, _DIM), jnp.float32)
    out_spec = pl.BlockSpec((_BE // 8, 8, _DIM), lambda i: (i, 0, 0))
    return pl.pallas_call(
        _edge23_body,
        grid=(nb,),
        in_specs=[pl.BlockSpec((_BE, _CH), lambda i: (i, 0))] + wspec + wspec,
        out_specs=[out_spec, out_spec],
        out_shape=[out_sds, out_sds],
    )(ea, *wsa, *wsb)


def _node_body(nchunk, kin, kmid, xt, parts, w1t, b1, w2t, b2, epsp1, hpre,
               stats):
    del kin, kmid
    i = pl.program_id(0)
    nb = pl.num_programs(0)
    x = xt[...]
    agg = parts[0] + parts[1]
    h0 = epsp1[0, 0] * x + agg
    h1 = jnp.maximum(jnp.dot(h0, w1t[...],
                             preferred_element_type=jnp.float32) + b1[...], 0.0)
    y = jnp.maximum(jnp.dot(h1, w2t[...],
                            preferred_element_type=jnp.float32) + b2[...], 0.0)
    hpre[...] = y

    @pl.when(i == 0)
    def _init():
        c = jnp.mean(y, axis=0, keepdims=True)
        stats[2:3, :] = c
        yc = y - c
        stats[0:1, :] = jnp.sum(yc, axis=0, keepdims=True)
        stats[1:2, :] = jnp.sum(yc * yc, axis=0, keepdims=True)

    @pl.when(i > 0)
    def _accum():
        c = stats[2:3, :]
        rowid = i * _BN + lax.broadcasted_iota(jnp.int32, (_BN, 1), 0)
        yc = jnp.where(rowid < _N, y - c, 0.0)
        stats[0:1, :] = stats[0:1, :] + jnp.sum(yc, axis=0, keepdims=True)
        stats[1:2, :] = stats[1:2, :] + jnp.sum(yc * yc, axis=0,
                                                keepdims=True)
    del nb


def _node_call(nchunk, kin, kmid, xt, parts, w1t, b1, w2t, b2, epsp1):
    xt_spec = pl.BlockSpec((_BN, nchunk * _CH), lambda i: (i, 0))
    return pl.pallas_call(
        functools.partial(_node_body, nchunk, kin, kmid),
        grid=(_NBLK,),
        in_specs=[
            xt_spec,
            pl.BlockSpec((_NC, _BN, nchunk * _CH), lambda i: (0, i, 0)),
            pl.BlockSpec((kin, kmid), lambda i: (0, 0)),
            pl.BlockSpec((1, kmid), lambda i: (0, 0)),
            pl.BlockSpec((kmid, _DIM), lambda i: (0, 0)),
            pl.BlockSpec((1, _DIM), lambda i: (0, 0)),
            pl.BlockSpec((1, 1), lambda i: (0, 0), memory_space=pltpu.SMEM),
        ],
        out_specs=[
            pl.BlockSpec((_BN, _DIM), lambda i: (i, 0)),
            pl.BlockSpec((3, _DIM), lambda i: (0, 0)),
        ],
        out_shape=[
            jax.ShapeDtypeStruct((_NR, _DIM), jnp.float32),
            jax.ShapeDtypeStruct((3, _DIM), jnp.float32),
        ],
    )(xt, parts, w1t, b1, w2t, b2, epsp1)


def _bn_body(hpre, stats, g, b, out):
    i = pl.program_id(0)
    d = stats[0:1, :] / _N
    mean = stats[2:3, :] + d
    var = stats[1:2, :] / _N - d * d
    y = (hpre[...] - mean) * lax.rsqrt(var + 1e-5) * g[...] + b[...]
    rowid = i * _BN + lax.broadcasted_iota(jnp.int32, (_BN, 1), 0)
    out[...] = jnp.where(rowid < _N, y, 0.0)


def _bn_call(hpre, stats, g, b):
    return pl.pallas_call(
        _bn_body,
        grid=(_NBLK,),
        in_specs=[
            pl.BlockSpec((_BN, _DIM), lambda i: (i, 0)),
            pl.BlockSpec((3, _DIM), lambda i: (0, 0)),
            pl.BlockSpec((1, _DIM), lambda i: (0, 0)),
            pl.BlockSpec((1, _DIM), lambda i: (0, 0)),
        ],
        out_specs=pl.BlockSpec((_BN, _DIM), lambda i: (i, 0)),
        out_shape=jax.ShapeDtypeStruct((_NR, _DIM), jnp.float32),
    )(hpre, stats, g, b)


def _fc_body(pp1, pp2, pp3, pp4, b2d, w1t, b1, w2t, b2, w3t, b3, w4t, b4, out):
    iota_g = lax.broadcasted_iota(jnp.int32, (_G, _BN), 0)

    def cnt_step(j, c):
        blk = b2d[pl.ds(j, 1), :]
        return c + jnp.sum((blk == iota_g).astype(jnp.float32), axis=1,
                           keepdims=True)

    cnt = lax.fori_loop(0, _NBLK, cnt_step,
                        jnp.zeros((_G, 1), jnp.float32))
    denom = jnp.maximum(cnt, 1.0)
    cols = []
    for pp in (pp1, pp2, pp3, pp4):
        for c in range(_NCH):
            cols.append(pp[0, c * _GS:c * _GS + _G, :]
                        + pp[1, c * _GS:c * _GS + _G, :])
    pooled = jnp.concatenate(cols, axis=-1) / denom
    h = jnp.maximum(jnp.dot(pooled, w1t[...],
                            preferred_element_type=jnp.float32) + b1[...], 0.0)
    h = jnp.maximum(jnp.dot(h, w2t[...],
                            preferred_element_type=jnp.float32) + b2[...], 0.0)
    h = jnp.maximum(jnp.dot(h, w3t[...],
                            preferred_element_type=jnp.float32) + b3[...], 0.0)
    out[...] = jnp.dot(h, w4t[...],
                       preferred_element_type=jnp.float32) + b4[...]


def _fc_call(pps, b2d, fcw):
    arows = _NCH * _GS
    pspec = pl.BlockSpec((_NC, arows, _CH), lambda: (0, 0, 0))
    return pl.pallas_call(
        _fc_body,
        grid=(),
        in_specs=[pspec, pspec, pspec, pspec,
                  pl.BlockSpec((_NBLK, _BN), lambda: (0, 0)),
                  pl.BlockSpec((4 * _DIM, _DIM), lambda: (0, 0)),
                  pl.BlockSpec((1, _DIM), lambda: (0, 0)),
                  pl.BlockSpec((_DIM, _DIM), lambda: (0, 0)),
                  pl.BlockSpec((1, _DIM), lambda: (0, 0)),
                  pl.BlockSpec((_DIM, _DIM), lambda: (0, 0)),
                  pl.BlockSpec((1, _DIM), lambda: (0, 0)),
                  pl.BlockSpec((_DIM, 128), lambda: (0, 0)),
                  pl.BlockSpec((1, 128), lambda: (0, 0))],
        out_specs=pl.BlockSpec((_G, 128), lambda: (0, 0)),
        out_shape=jax.ShapeDtypeStruct((_G, 128), jnp.float32),
    )(*pps, b2d, *fcw)


def _pad2(w, r, c):
    return jnp.pad(w, ((0, r - w.shape[0]), (0, c - w.shape[1])))


def _prep_edge_w(p, d1):
    w1t = _pad2(p['be_w1'].T, _CH, d1)
    b1 = _pad2(p['be_b1'][None, :], 1, d1)
    w2t = _pad2(p['be_w2'].T, d1, d1)
    b2 = _pad2(p['be_b2'][None, :], 1, d1)
    return w1t, b1, w2t, b2


def kernel(x, edge_index, edge_attr, batch, params):
    p = params
    src = edge_index[0]
    dst = edge_index[1]
    pe = _EPAD - _E
    pad_src = _N + (jnp.arange(pe, dtype=jnp.int32) % 64)
    pad_dst = _N + (jnp.arange(pe, dtype=jnp.int32) % 128)
    src_p = jnp.concatenate([src, pad_src])
    dst_p = jnp.concatenate([dst, pad_dst])
    src1 = src_p.reshape(_EPAD // 128, 128)
    offs = (jnp.arange(_NCH, dtype=jnp.int32) * _NR)[:, None]
    src8 = (src_p[None, :] + offs).reshape(_NCH * _EPAD // 128, 128)
    dstr = dst_p.reshape(_EPAD // 128, 128)

    batch_p = jnp.concatenate(
        [batch, _G + (jnp.arange(_NR - _N, dtype=jnp.int32) % 16)])
    goffs = (jnp.arange(_NCH, dtype=jnp.int32) * _GS)[:, None]
    b8 = (batch_p[None, :] + goffs).reshape(_NCH * _NR // 128, 128)
    b2d = batch_p.reshape(_NBLK, _BN)

    ea = jnp.pad(edge_attr, ((0, pe), (0, _CH - _NF)))
    xt0 = jnp.pad(x, ((0, _NR - _N), (0, _CH - _NF)))

    e1 = _edge1_call(ea, *_prep_edge_w(p['conv1'], _CH))
    e2c, e3c = _edge23_call(ea, _prep_edge_w(p['conv2'], _DIM),
                            _prep_edge_w(p['conv3'], _DIM))

    scat1 = _make_scatter(1)
    scat8 = _make_scatter(_NCH)
    pool = _make_pool()

    def conv_w(cp, kin, kmid):
        w1t = _pad2(cp['mlp_w1'].T, kin, kmid)
        b1 = _pad2(cp['mlp_b1'][None, :], 1, kmid)
        w2t = _pad2(cp['mlp_w2'].T, kmid, _DIM)
        b2 = cp['mlp_b2'][None, :]
        epsp1 = (1.0 + cp['eps']).reshape(1, 1)
        return w1t, b1, w2t, b2, epsp1

    parts1 = scat1(xt0, e1, src1, dstr)
    hp1, st1 = _node_call(1, _CH, _CH, xt0, parts1,
                          *conv_w(p['conv1'], _CH, _CH))
    xt1 = _bn_call(hp1, st1, p['bn1_g'][None, :], p['bn1_b'][None, :])

    def packed(xt_l):
        return jnp.transpose(xt_l.reshape(_NR, _NCH, _CH),
                             (1, 0, 2)).reshape(_NCH * _NR, _CH)

    xt1p = packed(xt1)
    pp1 = pool(xt1p, b8)

    def layer(xt_l, xt_lp, ec, cp, g, b):
        parts = scat8(xt_lp, ec, src8, dstr)
        hp, st = _node_call(_NCH, _DIM, _DIM, xt_l, parts,
                            *conv_w(cp, _DIM, _DIM))
        xt_n = _bn_call(hp, st, g[None, :], b[None, :])
        xt_np = packed(xt_n)
        return xt_n, xt_np, pool(xt_np, b8)

    xt2, xt2p, pp2 = layer(xt1, xt1p, e2c, p['conv2'], p['bn2_g'], p['bn2_b'])
    xt3, xt3p, pp3 = layer(xt2, xt2p, e3c, p['conv3'], p['bn3_g'], p['bn3_b'])
    _, _, pp4 = layer(xt3, xt3p, e3c, p['conv3'], p['bn4_g'], p['bn4_b'])

    fcw = (p['fc1_w'].T, p['fc1_b'][None, :],
           p['fc2_w'].T, p['fc2_b'][None, :],
           p['fc3_w'].T, p['fc3_b'][None, :],
           _pad2(p['fc4_w'].T, _DIM, 128), _pad2(p['fc4_b'][None, :], 1, 128))
    out = _fc_call([pp1, pp2, pp3, pp4], b2d, fcw)
    return out[:, 0]


# speedup vs baseline: 1.3474x; 1.1202x over previous
import functools

import jax
import jax.numpy as jnp
from jax import lax
from jax.experimental import pallas as pl
from jax.experimental.pallas import tpu as pltpu
from jax.experimental.pallas import tpu_sc as plsc

_N = 50000
_E = 800000
_G = 256
_NF = 21
_DIM = 256

_NC = 2
_NS = 16
_NW = _NC * _NS

_CH = 32
_NCH = _DIM // _CH
_NR = 50176
_W = 1024
_KI = _W // 128
_NWIN = 25
_EW = _NWIN * _W
_EPAD = _NW * _EW
_GS = 272
_PGRP = _NR // 1024
_BN = 512
_NBLK = _NR // _BN
_BE = 2048


def _mesh():
    return plsc.VectorSubcoreMesh(core_axis_name="c", subcore_axis_name="s",
                                  num_cores=_NC, num_subcores=_NS)


def _make_scatter(nchunk):
    acc_rows_per_tile = _NR // _NS
    n_zero = acc_rows_per_tile // 64
    _Q = _W // 4

    def body(xt, e, srcn, dstr, parts, acc, sidx, didx, xga, xgb, xgc, eb,
             eb2, sga, sgb, sgc, ssa, ssb, ssc):
        cid = lax.axis_index("c")
        sid = lax.axis_index("s")
        wid = sid * _NC + cid
        bufs = (xga, xgb, xgc)
        gsems = (sga, sgb, sgc)
        ssems = (ssa, ssb, ssc)

        @pl.loop(0, nchunk)
        def _chunk(ch):
            @pl.loop(0, 64)
            def _fill_z(r):
                z = jnp.zeros((16,), jnp.float32)
                xga[r, 0:16] = z
                xga[r, 16:32] = z

            @pl.loop(0, n_zero)
            def _zero(i):
                pltpu.sync_copy(
                    xga.at[pl.ds(0, 64)],
                    acc.at[pl.ds(pl.multiple_of(
                        sid * acc_rows_per_tile + i * 64, 64), 64)])

            plsc.subcore_barrier()

            @pl.loop(0, _NWIN)
            def _win(w):
                base = pl.multiple_of(wid * _EW + w * _W, _W)
                r0 = pl.multiple_of(ch * (_EPAD // 128) + base // 128, 8)
                pltpu.sync_copy(srcn.at[pl.ds(r0, _KI)], sidx)
                pltpu.sync_copy(dstr.at[pl.ds(pl.multiple_of(base // 128, 8),
                                              _KI)], didx)

                def gather(u):
                    b = u % 3
                    return pltpu.async_copy(
                        xt.at[sidx.at[u]], bufs[b], gsems[b])

                gd = {0: gather(0)}
                sd = {}
                for u in range(8):
                    b = u % 3
                    gd.pop(u).wait()
                    if u < 7:
                        nb = (u + 1) % 3
                        if nb in sd:
                            sd.pop(nb).wait()
                        gd[u + 1] = gather(u + 1)
                    if nchunk == 1:
                        if u % 2 == 0:
                            pltpu.sync_copy(
                                e.at[pl.ds(pl.multiple_of(
                                    (base + u * 128) // 8, 32), 32),
                                     slice(None), slice(None)], eb)
                        eof = 16 * (u % 2)
                    else:
                        pltpu.sync_copy(
                            e.at[pl.ds(pl.multiple_of(base + u * 128, 128),
                                       128),
                                 ch // 4,
                                 pl.ds((ch % 4) * _CH, _CH)], eb2)
                        eof = 0
                    xg = bufs[b]

                    @pl.loop(0, 16, unroll=4)
                    def _msg(o):
                        for q in range(8):
                            i = o * 8 + q
                            if nchunk == 1:
                                ea0 = eb[eof + o, q, 0:16]
                                ea1 = eb[eof + o, q, 16:32]
                            else:
                                ea0 = eb2[i, 0:16]
                                ea1 = eb2[i, 16:32]
                            a = xg[i, 0:16] + ea0
                            xg[i, 0:16] = jnp.maximum(a, 0.0)
                            bb = xg[i, 16:32] + ea1
                            xg[i, 16:32] = jnp.maximum(bb, 0.0)

                    sd[b] = pltpu.async_copy(xg, acc.at[didx.at[u]],
                                             ssems[b], add=True)
                for d in sd.values():
                    d.wait()

            plsc.subcore_barrier()

            row = pl.multiple_of(sid * acc_rows_per_tile, 64)
            pltpu.sync_copy(
                acc.at[pl.ds(row, acc_rows_per_tile)],
                parts.at[cid, pl.ds(row, acc_rows_per_tile),
                         pl.ds(ch * _CH, _CH)])

            plsc.subcore_barrier()

    return pl.kernel(
        body,
        out_type=jax.ShapeDtypeStruct((_NC, _NR, nchunk * _CH), jnp.float32),
        mesh=_mesh(),
        compiler_params=pltpu.CompilerParams(use_tc_tiling_on_sc=False),
        scratch_types=[
            pltpu.VMEM_SHARED((_NR, _CH), jnp.float32),
            pltpu.VMEM((_KI, 128), jnp.int32),
            pltpu.VMEM((_KI, 128), jnp.int32),
            pltpu.VMEM((128, _CH), jnp.float32),
            pltpu.VMEM((128, _CH), jnp.float32),
            pltpu.VMEM((128, _CH), jnp.float32),
            pltpu.VMEM((_Q // 8, 8, _CH), jnp.float32),
            pltpu.VMEM((128, _CH), jnp.float32),
            pltpu.SemaphoreType.DMA,
            pltpu.SemaphoreType.DMA,
            pltpu.SemaphoreType.DMA,
            pltpu.SemaphoreType.DMA,
            pltpu.SemaphoreType.DMA,
            pltpu.SemaphoreType.DMA,
        ],
    )


def _make_pool():
    arows = _NCH * _GS
    rows_per_tile = arows // _NS

    def body(xt, b8, pparts, acc, bidx, xb, zb, sem):
        del sem
        cid = lax.axis_index("c")
        sid = lax.axis_index("s")
        wid = sid * _NC + cid

        @pl.loop(0, rows_per_tile)
        def _fill_zb(r):
            z = jnp.zeros((16,), jnp.float32)
            zb[r, 0:16] = z
            zb[r, 16:32] = z

        pltpu.sync_copy(zb, acc.at[pl.ds(sid * rows_per_tile, rows_per_tile)])
        plsc.subcore_barrier()

        @pl.loop(0, _NCH)
        def _chunk(ch):
            @pl.loop(wid, _PGRP, step=_NW)
            def _grp(g):
                pltpu.sync_copy(b8.at[pl.ds(pl.multiple_of(
                    ch * (_NR // 128) + g * 8, 8), 8)], bidx)
                pltpu.sync_copy(xt.at[pl.ds(pl.multiple_of(
                    ch * _NR + g * 1024, 1024), 1024)], xb)
                for j in range(8):
                    pltpu.sync_copy(xb.at[pl.ds(j * 128, 128)],
                                    acc.at[bidx.at[j]], add=True)

        plsc.subcore_barrier()
        pltpu.sync_copy(acc.at[pl.ds(sid * rows_per_tile, rows_per_tile)],
                        pparts.at[cid, pl.ds(sid * rows_per_tile,
                                             rows_per_tile)])

    return pl.kernel(
        body,
        out_type=jax.ShapeDtypeStruct((_NC, arows, _CH), jnp.float32),
        mesh=_mesh(),
        compiler_params=pltpu.CompilerParams(use_tc_tiling_on_sc=False),
        scratch_types=[
            pltpu.VMEM_SHARED((arows, _CH), jnp.float32),
            pltpu.VMEM((8, 128), jnp.int32),
            pltpu.VMEM((1024, _CH), jnp.float32),
            pltpu.VMEM((rows_per_tile, _CH), jnp.float32),
            pltpu.SemaphoreType.DMA,
        ],
    )


def _edge1_body(ea, w1t, b1, w2t, b2, out):
    h = jnp.maximum(jnp.dot(ea[...], w1t[...],
                            preferred_element_type=jnp.float32) + b1[...], 0.0)
    ee = jnp.dot(h, w2t[...], preferred_element_type=jnp.float32) + b2[...]
    out[...] = ee.reshape(_BE // 8, 8, _CH)


def _edge1_call(ea, w1t, b1, w2t, b2):
    nb = _EPAD // _BE
    return pl.pallas_call(
        _edge1_body,
        grid=(nb,),
        in_specs=[
            pl.BlockSpec((_BE, _CH), lambda i: (i, 0)),
            pl.BlockSpec((_CH, _CH), lambda i: (0, 0)),
            pl.BlockSpec((1, _CH), lambda i: (0, 0)),
            pl.BlockSpec((_CH, _CH), lambda i: (0, 0)),
            pl.BlockSpec((1, _CH), lambda i: (0, 0)),
        ],
        out_specs=pl.BlockSpec((_BE // 8, 8, _CH), lambda i: (i, 0, 0)),
        out_shape=jax.ShapeDtypeStruct((_EPAD // 8, 8, _CH), jnp.float32),
    )(ea, w1t, b1, w2t, b2)


def _edge23_body(ea, w1a, b1a, w2a, b2a, w1b, b1b, w2b, b2b, oa, ob):
    x = ea[...]
    for (w1, b1, w2, b2, o) in ((w1a, b1a, w2a, b2a, oa),
                                (w1b, b1b, w2b, b2b, ob)):
        h = jnp.maximum(jnp.dot(x, w1[...],
                                preferred_element_type=jnp.float32) + b1[...],
                        0.0)
        ee = jnp.dot(h, w2[...], preferred_element_type=jnp.float32) + b2[...]
        o[:, 0, :] = ee[:, 0:128]
        o[:, 1, :] = ee[:, 128:256]


def _edge23_call(ea, wsa, wsb):
    nb = _EPAD // _BE
    wspec = [
        pl.BlockSpec((_CH, _DIM), lambda i: (0, 0)),
        pl.BlockSpec((1, _DIM), lambda i: (0, 0)),
        pl.BlockSpec((_DIM, _DIM), lambda i: (0, 0)),
        pl.BlockSpec((1, _DIM), lambda i: (0, 0)),
    ]
    out_sds = jax.ShapeDtypeStruct((_EPAD, 2, 128), jnp.float32)
    out_spec = pl.BlockSpec((_BE, 2, 128), lambda i: (i, 0, 0))
    return pl.pallas_call(
        _edge23_body,
        grid=(nb,),
        in_specs=[pl.BlockSpec((_BE, _CH), lambda i: (i, 0))] + wspec + wspec,
        out_specs=[out_spec, out_spec],
        out_shape=[out_sds, out_sds],
    )(ea, *wsa, *wsb)


def _node_body(nchunk, kin, kmid, xt, parts, w1t, b1, w2t, b2, epsp1, hpre,
               stats):
    del kin, kmid
    i = pl.program_id(0)
    nb = pl.num_programs(0)
    x = xt[...]
    agg = parts[0] + parts[1]
    h0 = epsp1[0, 0] * x + agg
    h1 = jnp.maximum(jnp.dot(h0, w1t[...],
                             preferred_element_type=jnp.float32) + b1[...], 0.0)
    y = jnp.maximum(jnp.dot(h1, w2t[...],
                            preferred_element_type=jnp.float32) + b2[...], 0.0)
    hpre[...] = y

    @pl.when(i == 0)
    def _init():
        c = jnp.mean(y, axis=0, keepdims=True)
        stats[2:3, :] = c
        yc = y - c
        stats[0:1, :] = jnp.sum(yc, axis=0, keepdims=True)
        stats[1:2, :] = jnp.sum(yc * yc, axis=0, keepdims=True)

    @pl.when(i > 0)
    def _accum():
        c = stats[2:3, :]
        rowid = i * _BN + lax.broadcasted_iota(jnp.int32, (_BN, 1), 0)
        yc = jnp.where(rowid < _N, y - c, 0.0)
        stats[0:1, :] = stats[0:1, :] + jnp.sum(yc, axis=0, keepdims=True)
        stats[1:2, :] = stats[1:2, :] + jnp.sum(yc * yc, axis=0,
                                                keepdims=True)
    del nb


def _node_call(nchunk, kin, kmid, xt, parts, w1t, b1, w2t, b2, epsp1):
    xt_spec = pl.BlockSpec((_BN, nchunk * _CH), lambda i: (i, 0))
    return pl.pallas_call(
        functools.partial(_node_body, nchunk, kin, kmid),
        grid=(_NBLK,),
        in_specs=[
            xt_spec,
            pl.BlockSpec((_NC, _BN, nchunk * _CH), lambda i: (0, i, 0)),
            pl.BlockSpec((kin, kmid), lambda i: (0, 0)),
            pl.BlockSpec((1, kmid), lambda i: (0, 0)),
            pl.BlockSpec((kmid, _DIM), lambda i: (0, 0)),
            pl.BlockSpec((1, _DIM), lambda i: (0, 0)),
            pl.BlockSpec((1, 1), lambda i: (0, 0), memory_space=pltpu.SMEM),
        ],
        out_specs=[
            pl.BlockSpec((_BN, _DIM), lambda i: (i, 0)),
            pl.BlockSpec((3, _DIM), lambda i: (0, 0)),
        ],
        out_shape=[
            jax.ShapeDtypeStruct((_NR, _DIM), jnp.float32),
            jax.ShapeDtypeStruct((3, _DIM), jnp.float32),
        ],
    )(xt, parts, w1t, b1, w2t, b2, epsp1)


def _bn_body(hpre, stats, g, b, out):
    i = pl.program_id(0)
    d = stats[0:1, :] / _N
    mean = stats[2:3, :] + d
    var = stats[1:2, :] / _N - d * d
    y = (hpre[...] - mean) * lax.rsqrt(var + 1e-5) * g[...] + b[...]
    rowid = i * _BN + lax.broadcasted_iota(jnp.int32, (_BN, 1), 0)
    out[...] = jnp.where(rowid < _N, y, 0.0)


def _bn_call(hpre, stats, g, b):
    return pl.pallas_call(
        _bn_body,
        grid=(_NBLK,),
        in_specs=[
            pl.BlockSpec((_BN, _DIM), lambda i: (i, 0)),
            pl.BlockSpec((3, _DIM), lambda i: (0, 0)),
            pl.BlockSpec((1, _DIM), lambda i: (0, 0)),
            pl.BlockSpec((1, _DIM), lambda i: (0, 0)),
        ],
        out_specs=pl.BlockSpec((_BN, _DIM), lambda i: (i, 0)),
        out_shape=jax.ShapeDtypeStruct((_NR, _DIM), jnp.float32),
    )(hpre, stats, g, b)


def _fc_body(pp1, pp2, pp3, pp4, b2d, w1t, b1, w2t, b2, w3t, b3, w4t, b4, out):
    iota_g = lax.broadcasted_iota(jnp.int32, (_G, _BN), 0)

    def cnt_step(j, c):
        blk = b2d[pl.ds(j, 1), :]
        return c + jnp.sum((blk == iota_g).astype(jnp.float32), axis=1,
                           keepdims=True)

    cnt = lax.fori_loop(0, _NBLK, cnt_step,
                        jnp.zeros((_G, 1), jnp.float32))
    denom = jnp.maximum(cnt, 1.0)
    cols = []
    for pp in (pp1, pp2, pp3, pp4):
        for c in range(_NCH):
            cols.append(pp[0, c * _GS:c * _GS + _G, :]
                        + pp[1, c * _GS:c * _GS + _G, :])
    pooled = jnp.concatenate(cols, axis=-1) / denom
    h = jnp.maximum(jnp.dot(pooled, w1t[...],
                            preferred_element_type=jnp.float32) + b1[...], 0.0)
    h = jnp.maximum(jnp.dot(h, w2t[...],
                            preferred_element_type=jnp.float32) + b2[...], 0.0)
    h = jnp.maximum(jnp.dot(h, w3t[...],
                            preferred_element_type=jnp.float32) + b3[...], 0.0)
    out[...] = jnp.dot(h, w4t[...],
                       preferred_element_type=jnp.float32) + b4[...]


def _fc_call(pps, b2d, fcw):
    arows = _NCH * _GS
    pspec = pl.BlockSpec((_NC, arows, _CH), lambda: (0, 0, 0))
    return pl.pallas_call(
        _fc_body,
        grid=(),
        in_specs=[pspec, pspec, pspec, pspec,
                  pl.BlockSpec((_NBLK, _BN), lambda: (0, 0)),
                  pl.BlockSpec((4 * _DIM, _DIM), lambda: (0, 0)),
                  pl.BlockSpec((1, _DIM), lambda: (0, 0)),
                  pl.BlockSpec((_DIM, _DIM), lambda: (0, 0)),
                  pl.BlockSpec((1, _DIM), lambda: (0, 0)),
                  pl.BlockSpec((_DIM, _DIM), lambda: (0, 0)),
                  pl.BlockSpec((1, _DIM), lambda: (0, 0)),
                  pl.BlockSpec((_DIM, 128), lambda: (0, 0)),
                  pl.BlockSpec((1, 128), lambda: (0, 0))],
        out_specs=pl.BlockSpec((_G, 128), lambda: (0, 0)),
        out_shape=jax.ShapeDtypeStruct((_G, 128), jnp.float32),
    )(*pps, b2d, *fcw)


def _pad2(w, r, c):
    return jnp.pad(w, ((0, r - w.shape[0]), (0, c - w.shape[1])))


def _prep_edge_w(p, d1):
    w1t = _pad2(p['be_w1'].T, _CH, d1)
    b1 = _pad2(p['be_b1'][None, :], 1, d1)
    w2t = _pad2(p['be_w2'].T, d1, d1)
    b2 = _pad2(p['be_b2'][None, :], 1, d1)
    return w1t, b1, w2t, b2


def kernel(x, edge_index, edge_attr, batch, params):
    p = params
    src = edge_index[0]
    dst = edge_index[1]
    pe = _EPAD - _E
    pad_src = _N + (jnp.arange(pe, dtype=jnp.int32) % 64)
    pad_dst = _N + (jnp.arange(pe, dtype=jnp.int32) % 128)
    src_p = jnp.concatenate([src, pad_src])
    dst_p = jnp.concatenate([dst, pad_dst])
    src1 = src_p.reshape(_EPAD // 128, 128)
    offs = (jnp.arange(_NCH, dtype=jnp.int32) * _NR)[:, None]
    src8 = (src_p[None, :] + offs).reshape(_NCH * _EPAD // 128, 128)
    dstr = dst_p.reshape(_EPAD // 128, 128)

    batch_p = jnp.concatenate(
        [batch, _G + (jnp.arange(_NR - _N, dtype=jnp.int32) % 16)])
    goffs = (jnp.arange(_NCH, dtype=jnp.int32) * _GS)[:, None]
    b8 = (batch_p[None, :] + goffs).reshape(_NCH * _NR // 128, 128)
    b2d = batch_p.reshape(_NBLK, _BN)

    ea = jnp.pad(edge_attr, ((0, pe), (0, _CH - _NF)))
    xt0 = jnp.pad(x, ((0, _NR - _N), (0, _CH - _NF)))

    e1 = _edge1_call(ea, *_prep_edge_w(p['conv1'], _CH))
    e2c, e3c = _edge23_call(ea, _prep_edge_w(p['conv2'], _DIM),
                            _prep_edge_w(p['conv3'], _DIM))

    scat1 = _make_scatter(1)
    scat8 = _make_scatter(_NCH)
    pool = _make_pool()

    def conv_w(cp, kin, kmid):
        w1t = _pad2(cp['mlp_w1'].T, kin, kmid)
        b1 = _pad2(cp['mlp_b1'][None, :], 1, kmid)
        w2t = _pad2(cp['mlp_w2'].T, kmid, _DIM)
        b2 = cp['mlp_b2'][None, :]
        epsp1 = (1.0 + cp['eps']).reshape(1, 1)
        return w1t, b1, w2t, b2, epsp1

    parts1 = scat1(xt0, e1, src1, dstr)
    hp1, st1 = _node_call(1, _CH, _CH, xt0, parts1,
                          *conv_w(p['conv1'], _CH, _CH))
    xt1 = _bn_call(hp1, st1, p['bn1_g'][None, :], p['bn1_b'][None, :])

    def packed(xt_l):
        return jnp.transpose(xt_l.reshape(_NR, _NCH, _CH),
                             (1, 0, 2)).reshape(_NCH * _NR, _CH)

    xt1p = packed(xt1)
    pp1 = pool(xt1p, b8)

    def layer(xt_l, xt_lp, ec, cp, g, b):
        parts = scat8(xt_lp, ec, src8, dstr)
        hp, st = _node_call(_NCH, _DIM, _DIM, xt_l, parts,
                            *conv_w(cp, _DIM, _DIM))
        xt_n = _bn_call(hp, st, g[None, :], b[None, :])
        xt_np = packed(xt_n)
        return xt_n, xt_np, pool(xt_np, b8)

    xt2, xt2p, pp2 = layer(xt1, xt1p, e2c, p['conv2'], p['bn2_g'], p['bn2_b'])
    xt3, xt3p, pp3 = layer(xt2, xt2p, e3c, p['conv3'], p['bn3_g'], p['bn3_b'])
    _, _, pp4 = layer(xt3, xt3p, e3c, p['conv3'], p['bn4_g'], p['bn4_b'])

    fcw = (p['fc1_w'].T, p['fc1_b'][None, :],
           p['fc2_w'].T, p['fc2_b'][None, :],
           p['fc3_w'].T, p['fc3_b'][None, :],
           _pad2(p['fc4_w'].T, _DIM, 128), _pad2(p['fc4_b'][None, :], 1, 128))
    out = _fc_call([pp1, pp2, pp3, pp4], b2d, fcw)
    return out[:, 0]


# speedup vs baseline: 1.4239x; 1.0568x over previous
import functools

import jax
import jax.numpy as jnp
from jax import lax
from jax.experimental import pallas as pl
from jax.experimental.pallas import tpu as pltpu
from jax.experimental.pallas import tpu_sc as plsc

_N = 50000
_E = 800000
_G = 256
_NF = 21
_DIM = 256

_NC = 2
_NS = 16
_NW = _NC * _NS

_CH = 32
_NCH = _DIM // _CH
_NR = 50176
_W = 1024
_KI = _W // 128
_NWIN = 25
_EW = _NWIN * _W
_EPAD = _NW * _EW
_GS = 272
_PGRP = _NR // 1024
_BN = 512
_NBLK = _NR // _BN
_BE = 2048


def _mesh():
    return plsc.VectorSubcoreMesh(core_axis_name="c", subcore_axis_name="s",
                                  num_cores=_NC, num_subcores=_NS)


def _make_scatter(nchunk):
    acc_rows_per_tile = _NR // _NS
    n_zero = acc_rows_per_tile // 64
    _Q = _W // 4

    def body(xt, e, srcn, dstr, parts, acc, sidx, didx, xga, xgb, xgc, eba,
             ebb, sga, sgb, sgc, ssa, ssb, ssc, sea, seb):
        cid = lax.axis_index("c")
        sid = lax.axis_index("s")
        wid = sid * _NC + cid
        bufs = (xga, xgb, xgc)
        gsems = (sga, sgb, sgc)
        ssems = (ssa, ssb, ssc)

        @pl.loop(0, nchunk)
        def _chunk(ch):
            @pl.loop(0, 64)
            def _fill_z(r):
                z = jnp.zeros((16,), jnp.float32)
                xga[r, 0:16] = z
                xga[r, 16:32] = z

            @pl.loop(0, n_zero)
            def _zero(i):
                pltpu.sync_copy(
                    xga.at[pl.ds(0, 64)],
                    acc.at[pl.ds(pl.multiple_of(
                        sid * acc_rows_per_tile + i * 64, 64), 64)])

            plsc.subcore_barrier()

            @pl.loop(0, _NWIN)
            def _win(w):
                base = pl.multiple_of(wid * _EW + w * _W, _W)
                r0 = pl.multiple_of(ch * (_EPAD // 128) + base // 128, 8)
                pltpu.sync_copy(srcn.at[pl.ds(r0, _KI)], sidx)
                pltpu.sync_copy(dstr.at[pl.ds(pl.multiple_of(base // 128, 8),
                                              _KI)], didx)

                def gather(u):
                    b = u % 3
                    return pltpu.async_copy(
                        xt.at[sidx.at[u]], bufs[b], gsems[b])

                ebufs = (eba, ebb)
                esems = (sea, seb)

                def eload(u):
                    r = pl.multiple_of(base + u * 128, 128)
                    if nchunk == 1:
                        src_e = e.at[pl.ds(r, 128)]
                    else:
                        src_e = e.at[pl.ds(r, 128), ch // 4,
                                     pl.ds((ch % 4) * _CH, _CH)]
                    return pltpu.async_copy(src_e, ebufs[u % 2],
                                            esems[u % 2])

                gd = {0: gather(0)}
                ed = {0: eload(0)}
                sd = {}
                for u in range(8):
                    b = u % 3
                    gd.pop(u).wait()
                    if u < 7:
                        nb = (u + 1) % 3
                        if nb in sd:
                            sd.pop(nb).wait()
                        gd[u + 1] = gather(u + 1)
                    if u < 7:
                        ed[u + 1] = eload(u + 1)
                    ed.pop(u).wait()
                    xg = bufs[b]
                    eb = ebufs[u % 2]

                    @pl.loop(0, 16, unroll=4)
                    def _msg(o):
                        for q in range(8):
                            i = o * 8 + q
                            a = xg[i, 0:16] + eb[i, 0:16]
                            xg[i, 0:16] = jnp.maximum(a, 0.0)
                            bb = xg[i, 16:32] + eb[i, 16:32]
                            xg[i, 16:32] = jnp.maximum(bb, 0.0)

                    sd[b] = pltpu.async_copy(xg, acc.at[didx.at[u]],
                                             ssems[b], add=True)
                for d in sd.values():
                    d.wait()

            plsc.subcore_barrier()

            row = pl.multiple_of(sid * acc_rows_per_tile, 64)
            pltpu.sync_copy(
                acc.at[pl.ds(row, acc_rows_per_tile)],
                parts.at[cid, pl.ds(row, acc_rows_per_tile),
                         pl.ds(ch * _CH, _CH)])

            plsc.subcore_barrier()

    return pl.kernel(
        body,
        out_type=jax.ShapeDtypeStruct((_NC, _NR, nchunk * _CH), jnp.float32),
        mesh=_mesh(),
        compiler_params=pltpu.CompilerParams(use_tc_tiling_on_sc=False),
        scratch_types=[
            pltpu.VMEM_SHARED((_NR, _CH), jnp.float32),
            pltpu.VMEM((_KI, 128), jnp.int32),
            pltpu.VMEM((_KI, 128), jnp.int32),
            pltpu.VMEM((128, _CH), jnp.float32),
            pltpu.VMEM((128, _CH), jnp.float32),
            pltpu.VMEM((128, _CH), jnp.float32),
            pltpu.VMEM((128, _CH), jnp.float32),
            pltpu.VMEM((128, _CH), jnp.float32),
            pltpu.SemaphoreType.DMA,
            pltpu.SemaphoreType.DMA,
            pltpu.SemaphoreType.DMA,
            pltpu.SemaphoreType.DMA,
            pltpu.SemaphoreType.DMA,
            pltpu.SemaphoreType.DMA,
            pltpu.SemaphoreType.DMA,
            pltpu.SemaphoreType.DMA,
        ],
    )


def _make_pool():
    arows = _NCH * _GS
    rows_per_tile = arows // _NS

    def body(xt, b8, pparts, acc, bidx, xb, zb, sem):
        del sem
        cid = lax.axis_index("c")
        sid = lax.axis_index("s")
        wid = sid * _NC + cid

        @pl.loop(0, rows_per_tile)
        def _fill_zb(r):
            z = jnp.zeros((16,), jnp.float32)
            zb[r, 0:16] = z
            zb[r, 16:32] = z

        pltpu.sync_copy(zb, acc.at[pl.ds(sid * rows_per_tile, rows_per_tile)])
        plsc.subcore_barrier()

        @pl.loop(0, _NCH)
        def _chunk(ch):
            @pl.loop(wid, _PGRP, step=_NW)
            def _grp(g):
                pltpu.sync_copy(b8.at[pl.ds(pl.multiple_of(
                    ch * (_NR // 128) + g * 8, 8), 8)], bidx)
                pltpu.sync_copy(xt.at[pl.ds(pl.multiple_of(
                    ch * _NR + g * 1024, 1024), 1024)], xb)
                for j in range(8):
                    pltpu.sync_copy(xb.at[pl.ds(j * 128, 128)],
                                    acc.at[bidx.at[j]], add=True)

        plsc.subcore_barrier()
        pltpu.sync_copy(acc.at[pl.ds(sid * rows_per_tile, rows_per_tile)],
                        pparts.at[cid, pl.ds(sid * rows_per_tile,
                                             rows_per_tile)])

    return pl.kernel(
        body,
        out_type=jax.ShapeDtypeStruct((_NC, arows, _CH), jnp.float32),
        mesh=_mesh(),
        compiler_params=pltpu.CompilerParams(use_tc_tiling_on_sc=False),
        scratch_types=[
            pltpu.VMEM_SHARED((arows, _CH), jnp.float32),
            pltpu.VMEM((8, 128), jnp.int32),
            pltpu.VMEM((1024, _CH), jnp.float32),
            pltpu.VMEM((rows_per_tile, _CH), jnp.float32),
            pltpu.SemaphoreType.DMA,
        ],
    )


def _edge1_body(ea, w1t, b1, w2t, b2, out):
    h = jnp.maximum(jnp.dot(ea[...], w1t[...],
                            preferred_element_type=jnp.float32) + b1[...], 0.0)
    out[...] = jnp.dot(h, w2t[...],
                       preferred_element_type=jnp.float32) + b2[...]


def _edge1_call(ea, w1t, b1, w2t, b2):
    nb = _EPAD // _BE
    return pl.pallas_call(
        _edge1_body,
        grid=(nb,),
        in_specs=[
            pl.BlockSpec((_BE, _CH), lambda i: (i, 0)),
            pl.BlockSpec((_CH, _CH), lambda i: (0, 0)),
            pl.BlockSpec((1, _CH), lambda i: (0, 0)),
            pl.BlockSpec((_CH, _CH), lambda i: (0, 0)),
            pl.BlockSpec((1, _CH), lambda i: (0, 0)),
        ],
        out_specs=pl.BlockSpec((_BE, _CH), lambda i: (i, 0)),
        out_shape=jax.ShapeDtypeStruct((_EPAD, _CH), jnp.float32),
    )(ea, w1t, b1, w2t, b2)


def _edge23_body(ea, w1a, b1a, w2a, b2a, w1b, b1b, w2b, b2b, oa, ob):
    x = ea[...]
    for (w1, b1, w2, b2, o) in ((w1a, b1a, w2a, b2a, oa),
                                (w1b, b1b, w2b, b2b, ob)):
        h = jnp.maximum(jnp.dot(x, w1[...],
                                preferred_element_type=jnp.float32) + b1[...],
                        0.0)
        ee = jnp.dot(h, w2[...], preferred_element_type=jnp.float32) + b2[...]
        o[:, 0, :] = ee[:, 0:128]
        o[:, 1, :] = ee[:, 128:256]


def _edge23_call(ea, wsa, wsb):
    nb = _EPAD // _BE
    wspec = [
        pl.BlockSpec((_CH, _DIM), lambda i: (0, 0)),
        pl.BlockSpec((1, _DIM), lambda i: (0, 0)),
        pl.BlockSpec((_DIM, _DIM), lambda i: (0, 0)),
        pl.BlockSpec((1, _DIM), lambda i: (0, 0)),
    ]
    out_sds = jax.ShapeDtypeStruct((_EPAD, 2, 128), jnp.float32)
    out_spec = pl.BlockSpec((_BE, 2, 128), lambda i: (i, 0, 0))
    return pl.pallas_call(
        _edge23_body,
        grid=(nb,),
        in_specs=[pl.BlockSpec((_BE, _CH), lambda i: (i, 0))] + wspec + wspec,
        out_specs=[out_spec, out_spec],
        out_shape=[out_sds, out_sds],
    )(ea, *wsa, *wsb)


def _node_body(nchunk, kin, kmid, xt, parts, w1t, b1, w2t, b2, epsp1, hpre,
               stats):
    del kin, kmid
    i = pl.program_id(0)
    nb = pl.num_programs(0)
    x = xt[...]
    agg = parts[0] + parts[1]
    h0 = epsp1[0, 0] * x + agg
    h1 = jnp.maximum(jnp.dot(h0, w1t[...],
                             preferred_element_type=jnp.float32) + b1[...], 0.0)
    y = jnp.maximum(jnp.dot(h1, w2t[...],
                            preferred_element_type=jnp.float32) + b2[...], 0.0)
    hpre[...] = y

    @pl.when(i == 0)
    def _init():
        c = jnp.mean(y, axis=0, keepdims=True)
        stats[2:3, :] = c
        yc = y - c
        stats[0:1, :] = jnp.sum(yc, axis=0, keepdims=True)
        stats[1:2, :] = jnp.sum(yc * yc, axis=0, keepdims=True)

    @pl.when(i > 0)
    def _accum():
        c = stats[2:3, :]
        rowid = i * _BN + lax.broadcasted_iota(jnp.int32, (_BN, 1), 0)
        yc = jnp.where(rowid < _N, y - c, 0.0)
        stats[0:1, :] = stats[0:1, :] + jnp.sum(yc, axis=0, keepdims=True)
        stats[1:2, :] = stats[1:2, :] + jnp.sum(yc * yc, axis=0,
                                                keepdims=True)
    del nb


def _node_call(nchunk, kin, kmid, xt, parts, w1t, b1, w2t, b2, epsp1):
    xt_spec = pl.BlockSpec((_BN, nchunk * _CH), lambda i: (i, 0))
    return pl.pallas_call(
        functools.partial(_node_body, nchunk, kin, kmid),
        grid=(_NBLK,),
        in_specs=[
            xt_spec,
            pl.BlockSpec((_NC, _BN, nchunk * _CH), lambda i: (0, i, 0)),
            pl.BlockSpec((kin, kmid), lambda i: (0, 0)),
            pl.BlockSpec((1, kmid), lambda i: (0, 0)),
            pl.BlockSpec((kmid, _DIM), lambda i: (0, 0)),
            pl.BlockSpec((1, _DIM), lambda i: (0, 0)),
            pl.BlockSpec((1, 1), lambda i: (0, 0), memory_space=pltpu.SMEM),
        ],
        out_specs=[
            pl.BlockSpec((_BN, _DIM), lambda i: (i, 0)),
            pl.BlockSpec((3, _DIM), lambda i: (0, 0)),
        ],
        out_shape=[
            jax.ShapeDtypeStruct((_NR, _DIM), jnp.float32),
            jax.ShapeDtypeStruct((3, _DIM), jnp.float32),
        ],
    )(xt, parts, w1t, b1, w2t, b2, epsp1)


def _bn_body(hpre, stats, g, b, out):
    i = pl.program_id(0)
    d = stats[0:1, :] / _N
    mean = stats[2:3, :] + d
    var = stats[1:2, :] / _N - d * d
    y = (hpre[...] - mean) * lax.rsqrt(var + 1e-5) * g[...] + b[...]
    rowid = i * _BN + lax.broadcasted_iota(jnp.int32, (_BN, 1), 0)
    out[...] = jnp.where(rowid < _N, y, 0.0)


def _bn_call(hpre, stats, g, b):
    return pl.pallas_call(
        _bn_body,
        grid=(_NBLK,),
        in_specs=[
            pl.BlockSpec((_BN, _DIM), lambda i: (i, 0)),
            pl.BlockSpec((3, _DIM), lambda i: (0, 0)),
            pl.BlockSpec((1, _DIM), lambda i: (0, 0)),
            pl.BlockSpec((1, _DIM), lambda i: (0, 0)),
        ],
        out_specs=pl.BlockSpec((_BN, _DIM), lambda i: (i, 0)),
        out_shape=jax.ShapeDtypeStruct((_NR, _DIM), jnp.float32),
    )(hpre, stats, g, b)


def _fc_body(pp1, pp2, pp3, pp4, b2d, w1t, b1, w2t, b2, w3t, b3, w4t, b4, out):
    iota_g = lax.broadcasted_iota(jnp.int32, (_G, _BN), 0)

    def cnt_step(j, c):
        blk = b2d[pl.ds(j, 1), :]
        return c + jnp.sum((blk == iota_g).astype(jnp.float32), axis=1,
                           keepdims=True)

    cnt = lax.fori_loop(0, _NBLK, cnt_step,
                        jnp.zeros((_G, 1), jnp.float32))
    denom = jnp.maximum(cnt, 1.0)
    cols = []
    for pp in (pp1, pp2, pp3, pp4):
        for c in range(_NCH):
            cols.append(pp[0, c * _GS:c * _GS + _G, :]
                        + pp[1, c * _GS:c * _GS + _G, :])
    pooled = jnp.concatenate(cols, axis=-1) / denom
    h = jnp.maximum(jnp.dot(pooled, w1t[...],
                            preferred_element_type=jnp.float32) + b1[...], 0.0)
    h = jnp.maximum(jnp.dot(h, w2t[...],
                            preferred_element_type=jnp.float32) + b2[...], 0.0)
    h = jnp.maximum(jnp.dot(h, w3t[...],
                            preferred_element_type=jnp.float32) + b3[...], 0.0)
    out[...] = jnp.dot(h, w4t[...],
                       preferred_element_type=jnp.float32) + b4[...]


def _fc_call(pps, b2d, fcw):
    arows = _NCH * _GS
    pspec = pl.BlockSpec((_NC, arows, _CH), lambda: (0, 0, 0))
    return pl.pallas_call(
        _fc_body,
        grid=(),
        in_specs=[pspec, pspec, pspec, pspec,
                  pl.BlockSpec((_NBLK, _BN), lambda: (0, 0)),
                  pl.BlockSpec((4 * _DIM, _DIM), lambda: (0, 0)),
                  pl.BlockSpec((1, _DIM), lambda: (0, 0)),
                  pl.BlockSpec((_DIM, _DIM), lambda: (0, 0)),
                  pl.BlockSpec((1, _DIM), lambda: (0, 0)),
                  pl.BlockSpec((_DIM, _DIM), lambda: (0, 0)),
                  pl.BlockSpec((1, _DIM), lambda: (0, 0)),
                  pl.BlockSpec((_DIM, 128), lambda: (0, 0)),
                  pl.BlockSpec((1, 128), lambda: (0, 0))],
        out_specs=pl.BlockSpec((_G, 128), lambda: (0, 0)),
        out_shape=jax.ShapeDtypeStruct((_G, 128), jnp.float32),
    )(*pps, b2d, *fcw)


def _pad2(w, r, c):
    return jnp.pad(w, ((0, r - w.shape[0]), (0, c - w.shape[1])))


def _prep_edge_w(p, d1):
    w1t = _pad2(p['be_w1'].T, _CH, d1)
    b1 = _pad2(p['be_b1'][None, :], 1, d1)
    w2t = _pad2(p['be_w2'].T, d1, d1)
    b2 = _pad2(p['be_b2'][None, :], 1, d1)
    return w1t, b1, w2t, b2


def kernel(x, edge_index, edge_attr, batch, params):
    p = params
    src = edge_index[0]
    dst = edge_index[1]
    pe = _EPAD - _E
    pad_src = _N + (jnp.arange(pe, dtype=jnp.int32) % 64)
    pad_dst = _N + (jnp.arange(pe, dtype=jnp.int32) % 128)
    src_p = jnp.concatenate([src, pad_src])
    dst_p = jnp.concatenate([dst, pad_dst])
    src1 = src_p.reshape(_EPAD // 128, 128)
    offs = (jnp.arange(_NCH, dtype=jnp.int32) * _NR)[:, None]
    src8 = (src_p[None, :] + offs).reshape(_NCH * _EPAD // 128, 128)
    dstr = dst_p.reshape(_EPAD // 128, 128)

    batch_p = jnp.concatenate(
        [batch, _G + (jnp.arange(_NR - _N, dtype=jnp.int32) % 16)])
    goffs = (jnp.arange(_NCH, dtype=jnp.int32) * _GS)[:, None]
    b8 = (batch_p[None, :] + goffs).reshape(_NCH * _NR // 128, 128)
    b2d = batch_p.reshape(_NBLK, _BN)

    ea = jnp.pad(edge_attr, ((0, pe), (0, _CH - _NF)))
    xt0 = jnp.pad(x, ((0, _NR - _N), (0, _CH - _NF)))

    e1 = _edge1_call(ea, *_prep_edge_w(p['conv1'], _CH))
    e2c, e3c = _edge23_call(ea, _prep_edge_w(p['conv2'], _DIM),
                            _prep_edge_w(p['conv3'], _DIM))

    scat1 = _make_scatter(1)
    scat8 = _make_scatter(_NCH)
    pool = _make_pool()

    def conv_w(cp, kin, kmid):
        w1t = _pad2(cp['mlp_w1'].T, kin, kmid)
        b1 = _pad2(cp['mlp_b1'][None, :], 1, kmid)
        w2t = _pad2(cp['mlp_w2'].T, kmid, _DIM)
        b2 = cp['mlp_b2'][None, :]
        epsp1 = (1.0 + cp['eps']).reshape(1, 1)
        return w1t, b1, w2t, b2, epsp1

    parts1 = scat1(xt0, e1, src1, dstr)
    hp1, st1 = _node_call(1, _CH, _CH, xt0, parts1,
                          *conv_w(p['conv1'], _CH, _CH))
    xt1 = _bn_call(hp1, st1, p['bn1_g'][None, :], p['bn1_b'][None, :])

    def packed(xt_l):
        return jnp.transpose(xt_l.reshape(_NR, _NCH, _CH),
                             (1, 0, 2)).reshape(_NCH * _NR, _CH)

    xt1p = packed(xt1)
    pp1 = pool(xt1p, b8)

    def layer(xt_l, xt_lp, ec, cp, g, b):
        parts = scat8(xt_lp, ec, src8, dstr)
        hp, st = _node_call(_NCH, _DIM, _DIM, xt_l, parts,
                            *conv_w(cp, _DIM, _DIM))
        xt_n = _bn_call(hp, st, g[None, :], b[None, :])
        xt_np = packed(xt_n)
        return xt_n, xt_np, pool(xt_np, b8)

    xt2, xt2p, pp2 = layer(xt1, xt1p, e2c, p['conv2'], p['bn2_g'], p['bn2_b'])
    xt3, xt3p, pp3 = layer(xt2, xt2p, e3c, p['conv3'], p['bn3_g'], p['bn3_b'])
    _, _, pp4 = layer(xt3, xt3p, e3c, p['conv3'], p['bn4_g'], p['bn4_b'])

    fcw = (p['fc1_w'].T, p['fc1_b'][None, :],
           p['fc2_w'].T, p['fc2_b'][None, :],
           p['fc3_w'].T, p['fc3_b'][None, :],
           _pad2(p['fc4_w'].T, _DIM, 128), _pad2(p['fc4_b'][None, :], 1, 128))
    out = _fc_call([pp1, pp2, pp3, pp4], b2d, fcw)
    return out[:, 0]
